# Initial kernel scaffold; baseline (speedup 1.0000x reference)
import jax
import jax.numpy as jnp
from jax.experimental import pallas as pl


def kernel(x, edge_index, params):
    raise NotImplementedError("write your pallas kernel here")


# speedup vs baseline: 5.2822x; 5.2822x over previous
import functools
import jax
import jax.numpy as jnp
from jax import lax
from jax.experimental import pallas as pl
from jax.experimental.pallas import tpu as pltpu
from jax.experimental.pallas import tpu_sc as plsc

N = 10000
E_RAW = 320000
E_TOT = E_RAW + N
D = 128
NEG = 0.2
NW = 32
CHUNK = 128
NCHUNK = 81
EPT = NCHUNK * CHUNK
EP = NW * EPT
NPAD = 10240
DEN_R = NPAD // D
ROWB = 1024
NROWB = NPAD // ROWB
DEN_RB = DEN_R // NROWB
HALF = NPAD // 2
ACC_R = HALF + CHUNK
SUB_R = ACC_R // 16
OUT_R = HALF // 16
DEN_H = HALF // D
DACC_R = DEN_H + 8

_mesh = plsc.VectorSubcoreMesh(core_axis_name="c", subcore_axis_name="s")


def _bfly_sum(v, lanes):
    for sh in (8, 4, 2, 1):
        v = v + jnp.take_along_axis(v, jnp.bitwise_xor(lanes, sh), axis=0)
    return v


def _bfly_max(v, lanes):
    for sh in (8, 4, 2, 1):
        v = jnp.maximum(
            v, jnp.take_along_axis(v, jnp.bitwise_xor(lanes, sh), axis=0))
    return v


def _den_column(d_blk):
    r = lax.broadcasted_iota(jnp.int32, (ROWB, DEN_RB), 0)
    k = lax.broadcasted_iota(jnp.int32, (ROWB, DEN_RB), 1)
    sel = (k == (r >> 7)).astype(jnp.float32)
    expanded = jnp.dot(sel, d_blk, preferred_element_type=jnp.float32)
    rr = lax.broadcasted_iota(jnp.int32, (ROWB, D), 0)
    cc = lax.broadcasted_iota(jnp.int32, (ROWB, D), 1)
    mask = (cc == (rr & (D - 1))).astype(jnp.float32)
    return jnp.sum(expanded * mask, axis=1, keepdims=True)


def _mm_first_body(h_ref, wl_ref, wr_ref, xl_ref, xr_ref):
    h = h_ref[...]
    xl_ref[...] = jnp.dot(h, wl_ref[...], preferred_element_type=jnp.float32)
    xr_ref[...] = jnp.dot(h, wr_ref[...], preferred_element_type=jnp.float32)


def _mm_first(h, wl, wr):
    return pl.pallas_call(
        _mm_first_body,
        grid=(NROWB,),
        in_specs=[
            pl.BlockSpec((ROWB, D), lambda i: (i, 0)),
            pl.BlockSpec((D, D), lambda i: (0, 0)),
            pl.BlockSpec((D, D), lambda i: (0, 0)),
        ],
        out_specs=[
            pl.BlockSpec((ROWB, D), lambda i: (i, 0)),
            pl.BlockSpec((ROWB, D), lambda i: (i, 0)),
        ],
        out_shape=[
            jax.ShapeDtypeStruct((NPAD, D), jnp.float32),
            jax.ShapeDtypeStruct((NPAD, D), jnp.float32),
        ],
    )(h, wl, wr)


def _combine_body(p0_ref, p1_ref, d0_ref, d1_ref, b_ref):
    den = _den_column(d0_ref[...] + d1_ref[...])
    return (p0_ref[...] + p1_ref[...]) / (den + 1e-16) + b_ref[...]


def _combine_mm_body(p0_ref, p1_ref, d0_ref, d1_ref, b_ref, wl_ref, wr_ref,
                     xl_ref, xr_ref):
    h = _combine_body(p0_ref, p1_ref, d0_ref, d1_ref, b_ref)
    h = jnp.maximum(h, 0.0)
    xl_ref[...] = jnp.dot(h, wl_ref[...], preferred_element_type=jnp.float32)
    xr_ref[...] = jnp.dot(h, wr_ref[...], preferred_element_type=jnp.float32)


def _combine_mm(p0, p1, d0, d1, b, wl, wr):
    return pl.pallas_call(
        _combine_mm_body,
        grid=(NROWB,),
        in_specs=[
            pl.BlockSpec((ROWB, D), lambda i: (i, 0)),
            pl.BlockSpec((ROWB, D), lambda i: (i, 0)),
            pl.BlockSpec((DEN_RB, D), lambda i: (i, 0)),
            pl.BlockSpec((DEN_RB, D), lambda i: (i, 0)),
            pl.BlockSpec((1, D), lambda i: (0, 0)),
            pl.BlockSpec((D, D), lambda i: (0, 0)),
            pl.BlockSpec((D, D), lambda i: (0, 0)),
        ],
        out_specs=[
            pl.BlockSpec((ROWB, D), lambda i: (i, 0)),
            pl.BlockSpec((ROWB, D), lambda i: (i, 0)),
        ],
        out_shape=[
            jax.ShapeDtypeStruct((NPAD, D), jnp.float32),
            jax.ShapeDtypeStruct((NPAD, D), jnp.float32),
        ],
    )(p0, p1, d0, d1, b, wl, wr)


def _final_body(p0_ref, p1_ref, d0_ref, d1_ref, b_ref, o_ref):
    i = pl.program_id(0)
    h = _combine_body(p0_ref, p1_ref, d0_ref, d1_ref, b_ref)
    gi = i * ROWB + lax.broadcasted_iota(jnp.int32, (ROWB, D), 0)
    h = jnp.where(gi < N, h, 0.0)
    s = jnp.sum(h, axis=0, keepdims=True) * (1.0 / N)

    @pl.when(i == 0)
    def _():
        o_ref[...] = s

    @pl.when(i > 0)
    def _():
        o_ref[...] += s


def _final_pool(p0, p1, d0, d1, b):
    return pl.pallas_call(
        _final_body,
        grid=(NROWB,),
        in_specs=[
            pl.BlockSpec((ROWB, D), lambda i: (i, 0)),
            pl.BlockSpec((ROWB, D), lambda i: (i, 0)),
            pl.BlockSpec((DEN_RB, D), lambda i: (i, 0)),
            pl.BlockSpec((DEN_RB, D), lambda i: (i, 0)),
            pl.BlockSpec((1, D), lambda i: (0, 0)),
        ],
        out_specs=pl.BlockSpec((1, D), lambda i: (0, 0)),
        out_shape=jax.ShapeDtypeStruct((1, D), jnp.float32),
    )(p0, p1, d0, d1, b)


@functools.partial(
    pl.kernel,
    out_type=[
        jax.ShapeDtypeStruct((EP,), jnp.float32),
        jax.ShapeDtypeStruct((NW, 16), jnp.float32),
    ],
    mesh=_mesh,
    scratch_types=[
        pltpu.VMEM((CHUNK,), jnp.int32),
        pltpu.VMEM((CHUNK,), jnp.int32),
        pltpu.VMEM((CHUNK, D), jnp.float32),
        pltpu.VMEM((CHUNK, D), jnp.float32),
        pltpu.VMEM((D,), jnp.float32),
        pltpu.VMEM((CHUNK,), jnp.float32),
        pltpu.VMEM((16,), jnp.float32),
        pltpu.SemaphoreType.DMA,
        pltpu.SemaphoreType.DMA,
    ],
)
def _sc_scores(xl_hbm, xr_hbm, att_hbm, src_hbm, dst_hbm, e_hbm, m_hbm,
               sidx, didx, xs, xr, attv, ebuf, mbuf, sem1, sem2):
    c = lax.axis_index("c")
    s = lax.axis_index("s")
    wid = s * 2 + c
    base = wid * EPT
    pltpu.sync_copy(att_hbm, attv)
    lanes = lax.iota(jnp.int32, 16)

    def chunk_body(ci, m16):
        off = base + ci * CHUNK
        pltpu.sync_copy(src_hbm.at[pl.ds(off, CHUNK)], sidx)
        pltpu.sync_copy(dst_hbm.at[pl.ds(off, CHUNK)], didx)
        cp1 = pltpu.async_copy(xl_hbm.at[sidx], xs, sem1)
        cp2 = pltpu.async_copy(xr_hbm.at[didx], xr, sem2)
        cp1.wait()
        cp2.wait()

        def group_body(g, m16i):
            e16 = jnp.zeros((16,), jnp.float32)
            for k in range(16):
                e = g * 16 + k
                acc = jnp.zeros((16,), jnp.float32)
                for j in range(8):
                    a = xs[e, pl.ds(j * 16, 16)] + xr[e, pl.ds(j * 16, 16)]
                    lk = jnp.maximum(a, NEG * a)
                    acc = acc + attv[pl.ds(j * 16, 16)] * lk
                e16 = jnp.where(lanes == k, _bfly_sum(acc, lanes), e16)
            ebuf[pl.ds(g * 16, 16)] = e16
            return jnp.maximum(m16i, e16)

        m16o = lax.fori_loop(0, CHUNK // 16, group_body, m16)
        pltpu.sync_copy(ebuf, e_hbm.at[pl.ds(off, CHUNK)])
        return m16o

    m16 = lax.fori_loop(0, NCHUNK, chunk_body,
                        jnp.full((16,), -3e38, jnp.float32))
    mbuf[...] = m16
    pltpu.sync_copy(mbuf, m_hbm.at[wid])


@functools.partial(
    pl.kernel,
    out_type=[
        jax.ShapeDtypeStruct((NPAD, D), jnp.float32),
        jax.ShapeDtypeStruct((NPAD, D), jnp.float32),
        jax.ShapeDtypeStruct((DEN_R, D), jnp.float32),
        jax.ShapeDtypeStruct((DEN_R, D), jnp.float32),
    ],
    mesh=_mesh,
    scratch_types=[
        pltpu.VMEM((CHUNK,), jnp.int32),
        pltpu.VMEM((CHUNK,), jnp.int32),
        pltpu.VMEM((CHUNK,), jnp.int32),
        pltpu.VMEM((CHUNK, D), jnp.float32),
        pltpu.VMEM((CHUNK, D), jnp.float32),
        pltpu.VMEM((CHUNK, D), jnp.float32),
        pltpu.VMEM((CHUNK,), jnp.float32),
        pltpu.VMEM((NW, 16), jnp.float32),
        pltpu.VMEM_SHARED((ACC_R, D), jnp.float32),
        pltpu.VMEM_SHARED((DEN_R, D), jnp.float32),
        pltpu.SemaphoreType.DMA,
    ],
)
def _sc_aggregate(xl_hbm, src_hbm, dst_hbm, e_hbm, m_hbm,
                  p0_hbm, p1_hbm, d0_hbm, d1_hbm,
                  sidx, didx, bidx, xs, scbuf, dnbuf, ebuf, mall,
                  accum, dacc, sem1):
    c = lax.axis_index("c")
    s = lax.axis_index("s")
    wid = s * 2 + c
    base = wid * EPT
    lanes = lax.iota(jnp.int32, 16)

    pltpu.sync_copy(m_hbm, mall)
    gm16 = mall[0, :]
    for i in range(1, NW):
        gm16 = jnp.maximum(gm16, mall[i, :])
    gm = _bfly_max(gm16, lanes)

    def zrow(e, _):
        for j in range(D // 16):
            scbuf[e, pl.ds(j * 16, 16)] = jnp.zeros((16,), jnp.float32)
        return 0

    def zero_accum():
        lax.fori_loop(0, CHUNK, zrow, 0)
        for k in range(2):
            pltpu.sync_copy(scbuf,
                            accum.at[pl.ds(s * SUB_R + k * CHUNK, CHUNK)])
        pltpu.sync_copy(
            scbuf.at[pl.ds(0, SUB_R - 2 * CHUNK)],
            accum.at[pl.ds(s * SUB_R + 2 * CHUNK, SUB_R - 2 * CHUNK)])

    zero_accum()

    @pl.when(s == 0)
    def _():
        pltpu.sync_copy(scbuf.at[pl.ds(0, DEN_R)], dacc)

    plsc.subcore_barrier()

    for half in range(2):
        nlo = half * HALF

        def chunk_body(ci, _):
            off = base + ci * CHUNK
            pltpu.sync_copy(src_hbm.at[pl.ds(off, CHUNK)], sidx)
            pltpu.sync_copy(dst_hbm.at[pl.ds(off, CHUNK)], didx)
            pltpu.sync_copy(e_hbm.at[pl.ds(off, CHUNK)], ebuf)
            pltpu.async_copy(xl_hbm.at[sidx], xs, sem1).wait()

            def group_body(g, _):
                ev = ebuf[pl.ds(g * 16, 16)]
                gi = off + g * 16 + lanes
                p16 = jnp.where(gi < E_TOT, jnp.exp(ev - gm), 0.0)
                dvg = didx[pl.ds(g * 16, 16)]
                dv = dvg - nlo
                mine = jnp.logical_and(dv >= 0, dv < HALF)
                didx[pl.ds(g * 16, 16)] = jnp.where(mine, dv, HALF)
                if half == 0:
                    bidx[pl.ds(g * 16, 16)] = dvg >> 7
                    dm = dvg & (D - 1)
                for k in range(16):
                    e = g * 16 + k
                    kk = jnp.full((16,), k, jnp.int32)
                    pvec = jnp.take_along_axis(p16, kk, axis=0)
                    for j in range(8):
                        scbuf[e, pl.ds(j * 16, 16)] = \
                            pvec * xs[e, pl.ds(j * 16, 16)]
                    if half == 0:
                        dmk = jnp.take_along_axis(dm, kk, axis=0)
                        for j in range(8):
                            dnbuf[e, pl.ds(j * 16, 16)] = \
                                jnp.where(lanes + (j * 16) == dmk, pvec, 0.0)
                return 0

            lax.fori_loop(0, CHUNK // 16, group_body, 0)
            pltpu.sync_copy(scbuf, accum.at[didx], add=True)
            if half == 0:
                pltpu.sync_copy(dnbuf, dacc.at[bidx], add=True)
            return 0

        lax.fori_loop(0, NCHUNK, chunk_body, 0)
        plsc.subcore_barrier()

        def copy_out(rlo, nrows):
            rows = accum.at[pl.ds(rlo, nrows)]

            @pl.when(c == 0)
            def _():
                pltpu.sync_copy(rows, p0_hbm.at[pl.ds(nlo + rlo, nrows)])

            @pl.when(c == 1)
            def _():
                pltpu.sync_copy(rows, p1_hbm.at[pl.ds(nlo + rlo, nrows)])

        for k in range(2):
            copy_out(s * OUT_R + k * CHUNK, CHUNK)
        copy_out(s * OUT_R + 2 * CHUNK, OUT_R - 2 * CHUNK)

        if half == 0:
            plsc.subcore_barrier()
            zero_accum()
            plsc.subcore_barrier()

    @pl.when(jnp.logical_and(s < DEN_R // 8, c == 0))
    def _():
        pltpu.sync_copy(dacc.at[pl.ds(s * 8, 8)], d0_hbm.at[pl.ds(s * 8, 8)])

    @pl.when(jnp.logical_and(s < DEN_R // 8, c == 1))
    def _():
        pltpu.sync_copy(dacc.at[pl.ds(s * 8, 8)], d1_hbm.at[pl.ds(s * 8, 8)])


def kernel(x, edge_index, params):
    sl = jnp.arange(N, dtype=jnp.int32)
    pad = jnp.zeros((EP - E_TOT,), jnp.int32)
    src = jnp.concatenate([edge_index[0].astype(jnp.int32), sl, pad])
    dst = jnp.concatenate([edge_index[1].astype(jnp.int32), sl, pad])
    xp = jnp.pad(x, ((0, NPAD - N), (0, 0)))

    nl = len(params) // 4
    p0 = p1 = d0 = d1 = bprev = None
    for i in range(nl):
        wl = params[f"Wl{i}"]
        wr = params[f"Wr{i}"]
        att = params[f"att{i}"]
        b = params[f"b{i}"].reshape(1, D)
        if i == 0:
            xl, xr = _mm_first(xp, wl, wr)
        else:
            xl, xr = _combine_mm(p0, p1, d0, d1, bprev, wl, wr)
        e, m = _sc_scores(xl, xr, att, src, dst)
        p0, p1, d0, d1 = _sc_aggregate(xl, src, dst, e, m)
        bprev = b
    return _final_pool(p0, p1, d0, d1, bprev)


# speedup vs baseline: 5.5969x; 1.0596x over previous
import functools
import jax
import jax.numpy as jnp
from jax import lax
from jax.experimental import pallas as pl
from jax.experimental.pallas import tpu as pltpu
from jax.experimental.pallas import tpu_sc as plsc

N = 10000
E_RAW = 320000
E_TOT = E_RAW + N
D = 128
NEG = 0.2
NW = 32
CHUNK = 128
NCHUNK = 81
EPT = NCHUNK * CHUNK
EP = NW * EPT
NPAD = 10240
DEN_R = NPAD // D
ROWB = 1024
NROWB = NPAD // ROWB
DEN_RB = DEN_R // NROWB
HALF = NPAD // 2
ACC_R = HALF + CHUNK
SUB_R = ACC_R // 16
OUT_R = HALF // 16
DEN_H = HALF // D
DACC_R = DEN_H + 8

_mesh = plsc.VectorSubcoreMesh(core_axis_name="c", subcore_axis_name="s")


def _bfly_sum(v, lanes):
    for sh in (8, 4, 2, 1):
        v = v + jnp.take_along_axis(v, jnp.bitwise_xor(lanes, sh), axis=0)
    return v


def _bfly_max(v, lanes):
    for sh in (8, 4, 2, 1):
        v = jnp.maximum(
            v, jnp.take_along_axis(v, jnp.bitwise_xor(lanes, sh), axis=0))
    return v


def _den_column(d_blk):
    r = lax.broadcasted_iota(jnp.int32, (ROWB, DEN_RB), 0)
    k = lax.broadcasted_iota(jnp.int32, (ROWB, DEN_RB), 1)
    sel = (k == (r >> 7)).astype(jnp.float32)
    expanded = jnp.dot(sel, d_blk, preferred_element_type=jnp.float32)
    rr = lax.broadcasted_iota(jnp.int32, (ROWB, D), 0)
    cc = lax.broadcasted_iota(jnp.int32, (ROWB, D), 1)
    mask = (cc == (rr & (D - 1))).astype(jnp.float32)
    return jnp.sum(expanded * mask, axis=1, keepdims=True)


def _mm_first_body(h_ref, wl_ref, wr_ref, xl_ref, xr_ref):
    h = h_ref[...]
    xl_ref[...] = jnp.dot(h, wl_ref[...], preferred_element_type=jnp.float32)
    xr_ref[...] = jnp.dot(h, wr_ref[...], preferred_element_type=jnp.float32)


def _mm_first(h, wl, wr):
    return pl.pallas_call(
        _mm_first_body,
        grid=(NROWB,),
        in_specs=[
            pl.BlockSpec((ROWB, D), lambda i: (i, 0)),
            pl.BlockSpec((D, D), lambda i: (0, 0)),
            pl.BlockSpec((D, D), lambda i: (0, 0)),
        ],
        out_specs=[
            pl.BlockSpec((ROWB, D), lambda i: (i, 0)),
            pl.BlockSpec((ROWB, D), lambda i: (i, 0)),
        ],
        out_shape=[
            jax.ShapeDtypeStruct((NPAD, D), jnp.float32),
            jax.ShapeDtypeStruct((NPAD, D), jnp.float32),
        ],
    )(h, wl, wr)


def _combine_body(p0_ref, p1_ref, d0_ref, d1_ref, b_ref):
    den = _den_column(d0_ref[...] + d1_ref[...])
    return (p0_ref[...] + p1_ref[...]) / (den + 1e-16) + b_ref[...]


def _combine_mm_body(p0_ref, p1_ref, d0_ref, d1_ref, b_ref, wl_ref, wr_ref,
                     xl_ref, xr_ref):
    h = _combine_body(p0_ref, p1_ref, d0_ref, d1_ref, b_ref)
    h = jnp.maximum(h, 0.0)
    xl_ref[...] = jnp.dot(h, wl_ref[...], preferred_element_type=jnp.float32)
    xr_ref[...] = jnp.dot(h, wr_ref[...], preferred_element_type=jnp.float32)


def _combine_mm(p0, p1, d0, d1, b, wl, wr):
    return pl.pallas_call(
        _combine_mm_body,
        grid=(NROWB,),
        in_specs=[
            pl.BlockSpec((ROWB, D), lambda i: (i, 0)),
            pl.BlockSpec((ROWB, D), lambda i: (i, 0)),
            pl.BlockSpec((DEN_RB, D), lambda i: (i, 0)),
            pl.BlockSpec((DEN_RB, D), lambda i: (i, 0)),
            pl.BlockSpec((1, D), lambda i: (0, 0)),
            pl.BlockSpec((D, D), lambda i: (0, 0)),
            pl.BlockSpec((D, D), lambda i: (0, 0)),
        ],
        out_specs=[
            pl.BlockSpec((ROWB, D), lambda i: (i, 0)),
            pl.BlockSpec((ROWB, D), lambda i: (i, 0)),
        ],
        out_shape=[
            jax.ShapeDtypeStruct((NPAD, D), jnp.float32),
            jax.ShapeDtypeStruct((NPAD, D), jnp.float32),
        ],
    )(p0, p1, d0, d1, b, wl, wr)


def _final_body(p0_ref, p1_ref, d0_ref, d1_ref, b_ref, o_ref):
    i = pl.program_id(0)
    h = _combine_body(p0_ref, p1_ref, d0_ref, d1_ref, b_ref)
    gi = i * ROWB + lax.broadcasted_iota(jnp.int32, (ROWB, D), 0)
    h = jnp.where(gi < N, h, 0.0)
    s = jnp.sum(h, axis=0, keepdims=True) * (1.0 / N)

    @pl.when(i == 0)
    def _():
        o_ref[...] = s

    @pl.when(i > 0)
    def _():
        o_ref[...] += s


def _final_pool(p0, p1, d0, d1, b):
    return pl.pallas_call(
        _final_body,
        grid=(NROWB,),
        in_specs=[
            pl.BlockSpec((ROWB, D), lambda i: (i, 0)),
            pl.BlockSpec((ROWB, D), lambda i: (i, 0)),
            pl.BlockSpec((DEN_RB, D), lambda i: (i, 0)),
            pl.BlockSpec((DEN_RB, D), lambda i: (i, 0)),
            pl.BlockSpec((1, D), lambda i: (0, 0)),
        ],
        out_specs=pl.BlockSpec((1, D), lambda i: (0, 0)),
        out_shape=jax.ShapeDtypeStruct((1, D), jnp.float32),
    )(p0, p1, d0, d1, b)


@functools.partial(
    pl.kernel,
    out_type=[
        jax.ShapeDtypeStruct((EP,), jnp.float32),
        jax.ShapeDtypeStruct((NW, 16), jnp.float32),
    ],
    mesh=_mesh,
    scratch_types=[
        pltpu.VMEM((CHUNK,), jnp.int32),
        pltpu.VMEM((CHUNK,), jnp.int32),
        pltpu.VMEM((CHUNK,), jnp.int32),
        pltpu.VMEM((CHUNK,), jnp.int32),
        pltpu.VMEM((CHUNK, D), jnp.float32),
        pltpu.VMEM((CHUNK, D), jnp.float32),
        pltpu.VMEM((CHUNK, D), jnp.float32),
        pltpu.VMEM((CHUNK, D), jnp.float32),
        pltpu.VMEM((CHUNK,), jnp.float32),
        pltpu.VMEM((D,), jnp.float32),
        pltpu.VMEM((16,), jnp.float32),
        pltpu.SemaphoreType.DMA,
        pltpu.SemaphoreType.DMA,
        pltpu.SemaphoreType.DMA,
        pltpu.SemaphoreType.DMA,
    ],
)
def _sc_scores(xl_hbm, xr_hbm, att_hbm, src_hbm, dst_hbm, e_hbm, m_hbm,
               si0, si1, dd0, dd1, xs0, xs1, xr0, xr1, ebuf, attv, mbuf,
               gp0, gp1, ix0, ix1):
    c = lax.axis_index("c")
    s = lax.axis_index("s")
    wid = s * 2 + c
    base = wid * EPT
    pltpu.sync_copy(att_hbm, attv)
    lanes = lax.iota(jnp.int32, 16)

    def start_idx(ci, si, dd, sem):
        cc = jnp.minimum(ci, NCHUNK - 1)
        off = base + cc * CHUNK
        pltpu.async_copy(src_hbm.at[pl.ds(off, CHUNK)], si, sem)
        pltpu.async_copy(dst_hbm.at[pl.ds(off, CHUNK)], dd, sem)

    def wait_idx(ci, si, dd, sem):
        cc = jnp.minimum(ci, NCHUNK - 1)
        off = base + cc * CHUNK
        pltpu.make_async_copy(src_hbm.at[pl.ds(off, CHUNK)], si, sem).wait()
        pltpu.make_async_copy(dst_hbm.at[pl.ds(off, CHUNK)], dd, sem).wait()

    def start_gather(si, dd, xs, xr, sem):
        pltpu.async_copy(xl_hbm.at[si], xs, sem)
        pltpu.async_copy(xr_hbm.at[dd], xr, sem)

    def wait_gather(si, dd, xs, xr, sem):
        pltpu.make_async_copy(xl_hbm.at[si], xs, sem).wait()
        pltpu.make_async_copy(xr_hbm.at[dd], xr, sem).wait()

    def compute(ci, xs, xr, m16):
        def group_body(g, m16i):
            def edge_body(k, e16i):
                e = g * 16 + k
                acc = jnp.zeros((16,), jnp.float32)
                for j in range(8):
                    a = xs[e, pl.ds(j * 16, 16)] + xr[e, pl.ds(j * 16, 16)]
                    lk = jnp.maximum(a, NEG * a)
                    acc = acc + attv[pl.ds(j * 16, 16)] * lk
                return jnp.where(lanes == k, _bfly_sum(acc, lanes), e16i)

            e16 = lax.fori_loop(0, 16, edge_body,
                                jnp.zeros((16,), jnp.float32))
            ebuf[pl.ds(g * 16, 16)] = e16
            return jnp.maximum(m16i, e16)

        m16 = lax.fori_loop(0, CHUNK // 16, group_body, m16)
        pltpu.sync_copy(ebuf, e_hbm.at[pl.ds(base + ci * CHUNK, CHUNK)])
        return m16

    def do_chunk(ci, si, dd, isem, xs, xr, gsem,
                 nsi, ndd, nisem, nxs, nxr, ngsem, m16, last=False):
        wait_gather(si, dd, xs, xr, gsem)
        if not last:
            wait_idx(ci + 1, nsi, ndd, nisem)
            start_gather(nsi, ndd, nxs, nxr, ngsem)
        m16 = compute(ci, xs, xr, m16)
        start_idx(ci + 2, si, dd, isem)
        return m16

    start_idx(0, si0, dd0, ix0)
    start_idx(1, si1, dd1, ix1)
    wait_idx(0, si0, dd0, ix0)
    start_gather(si0, dd0, xs0, xr0, gp0)

    def pair_body(t, m16):
        c0 = 2 * t
        m16 = do_chunk(c0, si0, dd0, ix0, xs0, xr0, gp0,
                       si1, dd1, ix1, xs1, xr1, gp1, m16)
        m16 = do_chunk(c0 + 1, si1, dd1, ix1, xs1, xr1, gp1,
                       si0, dd0, ix0, xs0, xr0, gp0, m16)
        return m16

    m16 = lax.fori_loop(0, (NCHUNK - 1) // 2, pair_body,
                        jnp.full((16,), -3e38, jnp.float32))
    m16 = do_chunk(NCHUNK - 1, si0, dd0, ix0, xs0, xr0, gp0,
                   si1, dd1, ix1, xs1, xr1, gp1, m16, last=True)
    wait_idx(NCHUNK - 1, si0, dd0, ix0)
    wait_idx(NCHUNK - 1, si1, dd1, ix1)
    mbuf[...] = m16
    pltpu.sync_copy(mbuf, m_hbm.at[wid])


@functools.partial(
    pl.kernel,
    out_type=[
        jax.ShapeDtypeStruct((NPAD, D), jnp.float32),
        jax.ShapeDtypeStruct((NPAD, D), jnp.float32),
        jax.ShapeDtypeStruct((DEN_R, D), jnp.float32),
        jax.ShapeDtypeStruct((DEN_R, D), jnp.float32),
    ],
    mesh=_mesh,
    scratch_types=[
        pltpu.VMEM((CHUNK,), jnp.int32),
        pltpu.VMEM((CHUNK,), jnp.int32),
        pltpu.VMEM((CHUNK,), jnp.int32),
        pltpu.VMEM((CHUNK,), jnp.int32),
        pltpu.VMEM((CHUNK,), jnp.float32),
        pltpu.VMEM((CHUNK,), jnp.float32),
        pltpu.VMEM((CHUNK,), jnp.int32),
        pltpu.VMEM((CHUNK,), jnp.int32),
        pltpu.VMEM((CHUNK,), jnp.int32),
        pltpu.VMEM((CHUNK, D), jnp.float32),
        pltpu.VMEM((CHUNK, D), jnp.float32),
        pltpu.VMEM((CHUNK, D), jnp.float32),
        pltpu.VMEM((CHUNK, D), jnp.float32),
        pltpu.VMEM((CHUNK, D), jnp.float32),
        pltpu.VMEM((NW, 16), jnp.float32),
        pltpu.VMEM_SHARED((ACC_R, D), jnp.float32),
        pltpu.VMEM_SHARED((DEN_R, D), jnp.float32),
        pltpu.SemaphoreType.DMA,
        pltpu.SemaphoreType.DMA,
        pltpu.SemaphoreType.DMA,
        pltpu.SemaphoreType.DMA,
        pltpu.SemaphoreType.DMA,
        pltpu.SemaphoreType.DMA,
        pltpu.SemaphoreType.DMA,
    ],
)
def _sc_aggregate(xl_hbm, src_hbm, dst_hbm, e_hbm, m_hbm,
                  p0_hbm, p1_hbm, d0_hbm, d1_hbm,
                  si0, si1, dd0, dd1, de0, de1, dl0, dl1, bidx,
                  xs0, xs1, sc0, sc1, dnbuf, mall, accum, dacc,
                  gx0, gx1, ss0, ss1, sd, ix0, ix1):
    c = lax.axis_index("c")
    s = lax.axis_index("s")
    wid = s * 2 + c
    base = wid * EPT
    lanes = lax.iota(jnp.int32, 16)

    pltpu.sync_copy(m_hbm, mall)
    gm16 = mall[0, :]
    for i in range(1, NW):
        gm16 = jnp.maximum(gm16, mall[i, :])
    gm = _bfly_max(gm16, lanes)

    def zero_accum(zsrc):
        for k in range(2):
            pltpu.sync_copy(zsrc,
                            accum.at[pl.ds(s * SUB_R + k * CHUNK, CHUNK)])
        pltpu.sync_copy(
            zsrc.at[pl.ds(0, SUB_R - 2 * CHUNK)],
            accum.at[pl.ds(s * SUB_R + 2 * CHUNK, SUB_R - 2 * CHUNK)])

    def start_idx(ci, si, dd, de, sem):
        cc = jnp.minimum(ci, NCHUNK - 1)
        off = base + cc * CHUNK
        pltpu.async_copy(src_hbm.at[pl.ds(off, CHUNK)], si, sem)
        pltpu.async_copy(dst_hbm.at[pl.ds(off, CHUNK)], dd, sem)
        pltpu.async_copy(e_hbm.at[pl.ds(off, CHUNK)], de, sem)

    def wait_idx(ci, si, dd, de, sem):
        cc = jnp.minimum(ci, NCHUNK - 1)
        off = base + cc * CHUNK
        pltpu.make_async_copy(src_hbm.at[pl.ds(off, CHUNK)], si, sem).wait()
        pltpu.make_async_copy(dst_hbm.at[pl.ds(off, CHUNK)], dd, sem).wait()
        pltpu.make_async_copy(e_hbm.at[pl.ds(off, CHUNK)], de, sem).wait()

    def start_gather(si, xs, sem):
        pltpu.async_copy(xl_hbm.at[si], xs, sem)

    def wait_gather(si, xs, sem):
        pltpu.make_async_copy(xl_hbm.at[si], xs, sem).wait()

    def start_scatter(sc, dl, sem):
        pltpu.async_copy(sc, accum.at[dl], sem, add=True)

    def wait_scatter(sc, dl, sem):
        pltpu.make_async_copy(sc, accum.at[dl], sem).wait()

    def start_dscatter():
        pltpu.async_copy(dnbuf, dacc.at[bidx], sd, add=True)

    def wait_dscatter():
        pltpu.make_async_copy(dnbuf, dacc.at[bidx], sd).wait()

    def init_idx(buf, val):
        def ib(g, _):
            buf[pl.ds(g * 16, 16)] = jnp.full((16,), val, jnp.int32)
            return 0
        lax.fori_loop(0, CHUNK // 16, ib, 0)

    def zero_rows(buf):
        def zr(e, _):
            for j in range(D // 16):
                buf[e, pl.ds(j * 16, 16)] = jnp.zeros((16,), jnp.float32)
            return 0
        lax.fori_loop(0, CHUNK, zr, 0)

    zero_rows(sc0)
    zero_rows(sc1)
    zero_rows(dnbuf)
    init_idx(dl0, HALF)
    init_idx(dl1, HALF)
    init_idx(bidx, 0)
    zero_accum(sc0)

    @pl.when(s == 0)
    def _():
        pltpu.sync_copy(sc0.at[pl.ds(0, DEN_R)], dacc)

    plsc.subcore_barrier()

    def compute(ci, xs, sc, dl, dd, de, half):
        lo = ci * CHUNK
        nlo = half * HALF

        def group_body(g, _):
            ev = de[pl.ds(g * 16, 16)]
            gi = base + lo + g * 16 + lanes
            p16 = jnp.where(gi < E_TOT, jnp.exp(ev - gm), 0.0)
            dvg = dd[pl.ds(g * 16, 16)]
            dv = dvg - nlo
            mine = jnp.logical_and(dv >= 0, dv < HALF)
            dl[pl.ds(g * 16, 16)] = jnp.where(mine, dv, HALF)
            if half == 0:
                bidx[pl.ds(g * 16, 16)] = dvg >> 7
            dm = dvg & (D - 1)

            def edge_body(k, _):
                e = g * 16 + k
                kk = jnp.full((16,), k, jnp.int32)
                pvec = jnp.take_along_axis(p16, kk, axis=0)
                for j in range(8):
                    sc[e, pl.ds(j * 16, 16)] = \
                        pvec * xs[e, pl.ds(j * 16, 16)]
                if half == 0:
                    dmk = jnp.take_along_axis(dm, kk, axis=0)
                    for j in range(8):
                        dnbuf[e, pl.ds(j * 16, 16)] = \
                            jnp.where(lanes + (j * 16) == dmk, pvec, 0.0)
                return 0

            lax.fori_loop(0, 16, edge_body, 0)
            return 0

        lax.fori_loop(0, CHUNK // 16, group_body, 0)

    for half in range(2):
        nlo = half * HALF
        start_scatter(sc0, dl0, ss0)
        start_scatter(sc1, dl1, ss1)
        if half == 0:
            start_dscatter()
        start_idx(0, si0, dd0, de0, ix0)
        start_idx(1, si1, dd1, de1, ix1)
        wait_idx(0, si0, dd0, de0, ix0)
        start_gather(si0, xs0, gx0)

        def do_chunk(ci, si, dd, de, isem, xs, gsem, sc, dl, ssem,
                     nsi, ndd, nde, nisem, nxs, ngsem, last=False):
            wait_gather(si, xs, gsem)
            if not last:
                wait_idx(ci + 1, nsi, ndd, nde, nisem)
                start_gather(nsi, nxs, ngsem)
            wait_scatter(sc, dl, ssem)
            if half == 0:
                wait_dscatter()
            compute(ci, xs, sc, dl, dd, de, half)
            start_idx(ci + 2, si, dd, de, isem)
            start_scatter(sc, dl, ssem)
            if half == 0:
                start_dscatter()

        def pair_body(t, _):
            c0 = 2 * t
            do_chunk(c0, si0, dd0, de0, ix0, xs0, gx0, sc0, dl0, ss0,
                     si1, dd1, de1, ix1, xs1, gx1)
            do_chunk(c0 + 1, si1, dd1, de1, ix1, xs1, gx1, sc1, dl1, ss1,
                     si0, dd0, de0, ix0, xs0, gx0)
            return 0

        lax.fori_loop(0, (NCHUNK - 1) // 2, pair_body, 0)
        do_chunk(NCHUNK - 1, si0, dd0, de0, ix0, xs0, gx0, sc0, dl0, ss0,
                 si1, dd1, de1, ix1, xs1, gx1, last=True)
        wait_idx(NCHUNK - 1, si0, dd0, de0, ix0)
        wait_idx(NCHUNK - 1, si1, dd1, de1, ix1)
        wait_scatter(sc0, dl0, ss0)
        wait_scatter(sc1, dl1, ss1)
        if half == 0:
            wait_dscatter()
        plsc.subcore_barrier()

        def copy_out(rlo, nrows):
            rows = accum.at[pl.ds(rlo, nrows)]

            @pl.when(c == 0)
            def _():
                pltpu.sync_copy(rows, p0_hbm.at[pl.ds(nlo + rlo, nrows)])

            @pl.when(c == 1)
            def _():
                pltpu.sync_copy(rows, p1_hbm.at[pl.ds(nlo + rlo, nrows)])

        for k in range(2):
            copy_out(s * OUT_R + k * CHUNK, CHUNK)
        copy_out(s * OUT_R + 2 * CHUNK, OUT_R - 2 * CHUNK)

        if half == 0:
            plsc.subcore_barrier()
            zero_rows(sc0)
            zero_rows(sc1)
            init_idx(dl0, HALF)
            init_idx(dl1, HALF)
            zero_accum(sc0)
            plsc.subcore_barrier()

    @pl.when(jnp.logical_and(s < DEN_R // 8, c == 0))
    def _():
        pltpu.sync_copy(dacc.at[pl.ds(s * 8, 8)], d0_hbm.at[pl.ds(s * 8, 8)])

    @pl.when(jnp.logical_and(s < DEN_R // 8, c == 1))
    def _():
        pltpu.sync_copy(dacc.at[pl.ds(s * 8, 8)], d1_hbm.at[pl.ds(s * 8, 8)])


def kernel(x, edge_index, params):
    sl = jnp.arange(N, dtype=jnp.int32)
    pad = jnp.zeros((EP - E_TOT,), jnp.int32)
    src = jnp.concatenate([edge_index[0].astype(jnp.int32), sl, pad])
    dst = jnp.concatenate([edge_index[1].astype(jnp.int32), sl, pad])
    xp = jnp.pad(x, ((0, NPAD - N), (0, 0)))

    nl = len(params) // 4
    p0 = p1 = d0 = d1 = bprev = None
    for i in range(nl):
        wl = params[f"Wl{i}"]
        wr = params[f"Wr{i}"]
        att = params[f"att{i}"]
        b = params[f"b{i}"].reshape(1, D)
        if i == 0:
            xl, xr = _mm_first(xp, wl, wr)
        else:
            xl, xr = _combine_mm(p0, p1, d0, d1, bprev, wl, wr)
        e, m = _sc_scores(xl, xr, att, src, dst)
        p0, p1, d0, d1 = _sc_aggregate(xl, src, dst, e, m)
        bprev = b
    return _final_pool(p0, p1, d0, d1, bprev)


# speedup vs baseline: 8.4916x; 1.5172x over previous
import functools
import jax
import jax.numpy as jnp
from jax import lax
from jax.experimental import pallas as pl
from jax.experimental.pallas import tpu as pltpu
from jax.experimental.pallas import tpu_sc as plsc

N = 10000
E_RAW = 320000
E_TOT = E_RAW + N
D = 128
NEG = 0.2
NW = 32
CHUNK = 128
NCHUNK = 81
EPT = NCHUNK * CHUNK
EP = NW * EPT
NPAD = 10240
DEN_R = NPAD // D
ROWB = 1024
NROWB = NPAD // ROWB
DEN_RB = DEN_R // NROWB
HALF = NPAD // 2
ACC_R = HALF + CHUNK
SUB_R = ACC_R // 16
OUT_R = HALF // 16
DEN_H = HALF // D
DACC_R = DEN_H + 8

_mesh = plsc.VectorSubcoreMesh(core_axis_name="c", subcore_axis_name="s")


def _bfly_sum(v, lanes):
    for sh in (8, 4, 2, 1):
        v = v + jnp.take_along_axis(v, jnp.bitwise_xor(lanes, sh), axis=0)
    return v


def _bfly_max(v, lanes):
    for sh in (8, 4, 2, 1):
        v = jnp.maximum(
            v, jnp.take_along_axis(v, jnp.bitwise_xor(lanes, sh), axis=0))
    return v


def _den_column(d_blk):
    r = lax.broadcasted_iota(jnp.int32, (ROWB, DEN_RB), 0)
    k = lax.broadcasted_iota(jnp.int32, (ROWB, DEN_RB), 1)
    sel = (k == (r >> 7)).astype(jnp.float32)
    expanded = jnp.dot(sel, d_blk, preferred_element_type=jnp.float32)
    rr = lax.broadcasted_iota(jnp.int32, (ROWB, D), 0)
    cc = lax.broadcasted_iota(jnp.int32, (ROWB, D), 1)
    mask = (cc == (rr & (D - 1))).astype(jnp.float32)
    return jnp.sum(expanded * mask, axis=1, keepdims=True)


def _mm_first_body(h_ref, wl_ref, wr_ref, xl_ref, xr_ref):
    h = h_ref[...]
    xl_ref[...] = jnp.dot(h, wl_ref[...], preferred_element_type=jnp.float32)
    xr_ref[...] = jnp.dot(h, wr_ref[...], preferred_element_type=jnp.float32)


def _mm_first(h, wl, wr):
    return pl.pallas_call(
        _mm_first_body,
        grid=(NROWB,),
        in_specs=[
            pl.BlockSpec((ROWB, D), lambda i: (i, 0)),
            pl.BlockSpec((D, D), lambda i: (0, 0)),
            pl.BlockSpec((D, D), lambda i: (0, 0)),
        ],
        out_specs=[
            pl.BlockSpec((ROWB, D), lambda i: (i, 0)),
            pl.BlockSpec((ROWB, D), lambda i: (i, 0)),
        ],
        out_shape=[
            jax.ShapeDtypeStruct((NPAD, D), jnp.float32),
            jax.ShapeDtypeStruct((NPAD, D), jnp.float32),
        ],
    )(h, wl, wr)


def _combine_body(p0_ref, p1_ref, d0_ref, d1_ref, b_ref):
    den = _den_column(d0_ref[...] + d1_ref[...])
    return (p0_ref[...] + p1_ref[...]) / (den + 1e-16) + b_ref[...]


def _combine_mm_body(p0_ref, p1_ref, d0_ref, d1_ref, b_ref, wl_ref, wr_ref,
                     xl_ref, xr_ref):
    h = _combine_body(p0_ref, p1_ref, d0_ref, d1_ref, b_ref)
    h = jnp.maximum(h, 0.0)
    xl_ref[...] = jnp.dot(h, wl_ref[...], preferred_element_type=jnp.float32)
    xr_ref[...] = jnp.dot(h, wr_ref[...], preferred_element_type=jnp.float32)


def _combine_mm(p0, p1, d0, d1, b, wl, wr):
    return pl.pallas_call(
        _combine_mm_body,
        grid=(NROWB,),
        in_specs=[
            pl.BlockSpec((ROWB, D), lambda i: (i, 0)),
            pl.BlockSpec((ROWB, D), lambda i: (i, 0)),
            pl.BlockSpec((DEN_RB, D), lambda i: (i, 0)),
            pl.BlockSpec((DEN_RB, D), lambda i: (i, 0)),
            pl.BlockSpec((1, D), lambda i: (0, 0)),
            pl.BlockSpec((D, D), lambda i: (0, 0)),
            pl.BlockSpec((D, D), lambda i: (0, 0)),
        ],
        out_specs=[
            pl.BlockSpec((ROWB, D), lambda i: (i, 0)),
            pl.BlockSpec((ROWB, D), lambda i: (i, 0)),
        ],
        out_shape=[
            jax.ShapeDtypeStruct((NPAD, D), jnp.float32),
            jax.ShapeDtypeStruct((NPAD, D), jnp.float32),
        ],
    )(p0, p1, d0, d1, b, wl, wr)


def _final_body(p0_ref, p1_ref, d0_ref, d1_ref, b_ref, o_ref):
    i = pl.program_id(0)
    h = _combine_body(p0_ref, p1_ref, d0_ref, d1_ref, b_ref)
    gi = i * ROWB + lax.broadcasted_iota(jnp.int32, (ROWB, D), 0)
    h = jnp.where(gi < N, h, 0.0)
    s = jnp.sum(h, axis=0, keepdims=True) * (1.0 / N)

    @pl.when(i == 0)
    def _():
        o_ref[...] = s

    @pl.when(i > 0)
    def _():
        o_ref[...] += s


def _final_pool(p0, p1, d0, d1, b):
    return pl.pallas_call(
        _final_body,
        grid=(NROWB,),
        in_specs=[
            pl.BlockSpec((ROWB, D), lambda i: (i, 0)),
            pl.BlockSpec((ROWB, D), lambda i: (i, 0)),
            pl.BlockSpec((DEN_RB, D), lambda i: (i, 0)),
            pl.BlockSpec((DEN_RB, D), lambda i: (i, 0)),
            pl.BlockSpec((1, D), lambda i: (0, 0)),
        ],
        out_specs=pl.BlockSpec((1, D), lambda i: (0, 0)),
        out_shape=jax.ShapeDtypeStruct((1, D), jnp.float32),
    )(p0, p1, d0, d1, b)


@functools.partial(
    pl.kernel,
    out_type=[
        jax.ShapeDtypeStruct((EP,), jnp.float32),
        jax.ShapeDtypeStruct((NW, 16), jnp.float32),
    ],
    mesh=_mesh,
    scratch_types=[
        pltpu.VMEM((CHUNK,), jnp.int32),
        pltpu.VMEM((CHUNK,), jnp.int32),
        pltpu.VMEM((CHUNK,), jnp.int32),
        pltpu.VMEM((CHUNK,), jnp.int32),
        pltpu.VMEM((CHUNK, D), jnp.float32),
        pltpu.VMEM((CHUNK, D), jnp.float32),
        pltpu.VMEM((CHUNK, D), jnp.float32),
        pltpu.VMEM((CHUNK, D), jnp.float32),
        pltpu.VMEM((CHUNK,), jnp.float32),
        pltpu.VMEM((D,), jnp.float32),
        pltpu.VMEM((16,), jnp.float32),
        pltpu.SemaphoreType.DMA,
        pltpu.SemaphoreType.DMA,
        pltpu.SemaphoreType.DMA,
        pltpu.SemaphoreType.DMA,
    ],
)
def _sc_scores(xl_hbm, xr_hbm, att_hbm, src_hbm, dst_hbm, e_hbm, m_hbm,
               si0, si1, dd0, dd1, xs0, xs1, xr0, xr1, ebuf, attv, mbuf,
               gp0, gp1, ix0, ix1):
    c = lax.axis_index("c")
    s = lax.axis_index("s")
    wid = s * 2 + c
    base = wid * EPT
    pltpu.sync_copy(att_hbm, attv)
    lanes = lax.iota(jnp.int32, 16)

    def start_idx(ci, si, dd, sem):
        cc = jnp.minimum(ci, NCHUNK - 1)
        off = base + cc * CHUNK
        pltpu.async_copy(src_hbm.at[pl.ds(off, CHUNK)], si, sem)
        pltpu.async_copy(dst_hbm.at[pl.ds(off, CHUNK)], dd, sem)

    def wait_idx(ci, si, dd, sem):
        cc = jnp.minimum(ci, NCHUNK - 1)
        off = base + cc * CHUNK
        pltpu.make_async_copy(src_hbm.at[pl.ds(off, CHUNK)], si, sem).wait()
        pltpu.make_async_copy(dst_hbm.at[pl.ds(off, CHUNK)], dd, sem).wait()

    def start_gather(si, dd, xs, xr, sem):
        pltpu.async_copy(xl_hbm.at[si], xs, sem)
        pltpu.async_copy(xr_hbm.at[dd], xr, sem)

    def wait_gather(si, dd, xs, xr, sem):
        pltpu.make_async_copy(xl_hbm.at[si], xs, sem).wait()
        pltpu.make_async_copy(xr_hbm.at[dd], xr, sem).wait()

    def compute(ci, xs, xr, m16):
        def group_body(g, m16i):
            @plsc.parallel_loop(0, 16, unroll=4,
                                carry=jnp.zeros((16,), jnp.float32))
            def e16(k, e16i):
                e = g * 16 + k
                acc = jnp.zeros((16,), jnp.float32)
                for j in range(8):
                    a = xs[e, pl.ds(j * 16, 16)] + xr[e, pl.ds(j * 16, 16)]
                    lk = jnp.maximum(a, NEG * a)
                    acc = acc + attv[pl.ds(j * 16, 16)] * lk
                return jnp.where(lanes == k, _bfly_sum(acc, lanes), e16i)

            ebuf[pl.ds(g * 16, 16)] = e16
            return jnp.maximum(m16i, e16)

        m16 = lax.fori_loop(0, CHUNK // 16, group_body, m16)
        pltpu.sync_copy(ebuf, e_hbm.at[pl.ds(base + ci * CHUNK, CHUNK)])
        return m16

    def do_chunk(ci, si, dd, isem, xs, xr, gsem,
                 nsi, ndd, nisem, nxs, nxr, ngsem, m16, last=False):
        wait_gather(si, dd, xs, xr, gsem)
        if not last:
            wait_idx(ci + 1, nsi, ndd, nisem)
            start_gather(nsi, ndd, nxs, nxr, ngsem)
        m16 = compute(ci, xs, xr, m16)
        start_idx(ci + 2, si, dd, isem)
        return m16

    start_idx(0, si0, dd0, ix0)
    start_idx(1, si1, dd1, ix1)
    wait_idx(0, si0, dd0, ix0)
    start_gather(si0, dd0, xs0, xr0, gp0)

    def pair_body(t, m16):
        c0 = 2 * t
        m16 = do_chunk(c0, si0, dd0, ix0, xs0, xr0, gp0,
                       si1, dd1, ix1, xs1, xr1, gp1, m16)
        m16 = do_chunk(c0 + 1, si1, dd1, ix1, xs1, xr1, gp1,
                       si0, dd0, ix0, xs0, xr0, gp0, m16)
        return m16

    m16 = lax.fori_loop(0, (NCHUNK - 1) // 2, pair_body,
                        jnp.full((16,), -3e38, jnp.float32))
    m16 = do_chunk(NCHUNK - 1, si0, dd0, ix0, xs0, xr0, gp0,
                   si1, dd1, ix1, xs1, xr1, gp1, m16, last=True)
    wait_idx(NCHUNK - 1, si0, dd0, ix0)
    wait_idx(NCHUNK - 1, si1, dd1, ix1)
    mbuf[...] = m16
    pltpu.sync_copy(mbuf, m_hbm.at[wid])


@functools.partial(
    pl.kernel,
    out_type=[
        jax.ShapeDtypeStruct((NPAD, D), jnp.float32),
        jax.ShapeDtypeStruct((NPAD, D), jnp.float32),
        jax.ShapeDtypeStruct((DEN_R, D), jnp.float32),
        jax.ShapeDtypeStruct((DEN_R, D), jnp.float32),
    ],
    mesh=_mesh,
    scratch_types=[
        pltpu.VMEM((CHUNK,), jnp.int32),
        pltpu.VMEM((CHUNK,), jnp.int32),
        pltpu.VMEM((CHUNK,), jnp.int32),
        pltpu.VMEM((CHUNK,), jnp.int32),
        pltpu.VMEM((CHUNK,), jnp.float32),
        pltpu.VMEM((CHUNK,), jnp.float32),
        pltpu.VMEM((CHUNK,), jnp.int32),
        pltpu.VMEM((CHUNK,), jnp.int32),
        pltpu.VMEM((CHUNK,), jnp.int32),
        pltpu.VMEM((CHUNK, D), jnp.float32),
        pltpu.VMEM((CHUNK, D), jnp.float32),
        pltpu.VMEM((CHUNK, D), jnp.float32),
        pltpu.VMEM((CHUNK, D), jnp.float32),
        pltpu.VMEM((CHUNK, D), jnp.float32),
        pltpu.VMEM((NW, 16), jnp.float32),
        pltpu.VMEM_SHARED((ACC_R, D), jnp.float32),
        pltpu.VMEM_SHARED((DEN_R, D), jnp.float32),
        pltpu.SemaphoreType.DMA,
        pltpu.SemaphoreType.DMA,
        pltpu.SemaphoreType.DMA,
        pltpu.SemaphoreType.DMA,
        pltpu.SemaphoreType.DMA,
        pltpu.SemaphoreType.DMA,
        pltpu.SemaphoreType.DMA,
    ],
)
def _sc_aggregate(xl_hbm, src_hbm, dst_hbm, e_hbm, m_hbm,
                  p0_hbm, p1_hbm, d0_hbm, d1_hbm,
                  si0, si1, dd0, dd1, de0, de1, dl0, dl1, bidx,
                  xs0, xs1, sc0, sc1, dnbuf, mall, accum, dacc,
                  gx0, gx1, ss0, ss1, sd, ix0, ix1):
    c = lax.axis_index("c")
    s = lax.axis_index("s")
    wid = s * 2 + c
    base = wid * EPT
    lanes = lax.iota(jnp.int32, 16)

    pltpu.sync_copy(m_hbm, mall)
    gm16 = mall[0, :]
    for i in range(1, NW):
        gm16 = jnp.maximum(gm16, mall[i, :])
    gm = _bfly_max(gm16, lanes)

    def zero_accum(zsrc):
        for k in range(2):
            pltpu.sync_copy(zsrc,
                            accum.at[pl.ds(s * SUB_R + k * CHUNK, CHUNK)])
        pltpu.sync_copy(
            zsrc.at[pl.ds(0, SUB_R - 2 * CHUNK)],
            accum.at[pl.ds(s * SUB_R + 2 * CHUNK, SUB_R - 2 * CHUNK)])

    def start_idx(ci, si, dd, de, sem):
        cc = jnp.minimum(ci, NCHUNK - 1)
        off = base + cc * CHUNK
        pltpu.async_copy(src_hbm.at[pl.ds(off, CHUNK)], si, sem)
        pltpu.async_copy(dst_hbm.at[pl.ds(off, CHUNK)], dd, sem)
        pltpu.async_copy(e_hbm.at[pl.ds(off, CHUNK)], de, sem)

    def wait_idx(ci, si, dd, de, sem):
        cc = jnp.minimum(ci, NCHUNK - 1)
        off = base + cc * CHUNK
        pltpu.make_async_copy(src_hbm.at[pl.ds(off, CHUNK)], si, sem).wait()
        pltpu.make_async_copy(dst_hbm.at[pl.ds(off, CHUNK)], dd, sem).wait()
        pltpu.make_async_copy(e_hbm.at[pl.ds(off, CHUNK)], de, sem).wait()

    def start_gather(si, xs, sem):
        pltpu.async_copy(xl_hbm.at[si], xs, sem)

    def wait_gather(si, xs, sem):
        pltpu.make_async_copy(xl_hbm.at[si], xs, sem).wait()

    def start_scatter(sc, dl, sem):
        pltpu.async_copy(sc, accum.at[dl], sem, add=True)

    def wait_scatter(sc, dl, sem):
        pltpu.make_async_copy(sc, accum.at[dl], sem).wait()

    def start_dscatter():
        pltpu.async_copy(dnbuf, dacc.at[bidx], sd, add=True)

    def wait_dscatter():
        pltpu.make_async_copy(dnbuf, dacc.at[bidx], sd).wait()

    def init_idx(buf, val):
        def ib(g, _):
            buf[pl.ds(g * 16, 16)] = jnp.full((16,), val, jnp.int32)
            return 0
        lax.fori_loop(0, CHUNK // 16, ib, 0)

    def zero_rows(buf):
        def zr(e, _):
            for j in range(D // 16):
                buf[e, pl.ds(j * 16, 16)] = jnp.zeros((16,), jnp.float32)
            return 0
        lax.fori_loop(0, CHUNK, zr, 0)

    zero_rows(sc0)
    zero_rows(sc1)
    zero_rows(dnbuf)
    init_idx(dl0, HALF)
    init_idx(dl1, HALF)
    init_idx(bidx, 0)
    zero_accum(sc0)

    @pl.when(s == 0)
    def _():
        pltpu.sync_copy(sc0.at[pl.ds(0, DEN_R)], dacc)

    plsc.subcore_barrier()

    def compute(ci, xs, sc, dl, dd, de, half):
        lo = ci * CHUNK
        nlo = half * HALF

        def group_body(g, _):
            ev = de[pl.ds(g * 16, 16)]
            gi = base + lo + g * 16 + lanes
            p16 = jnp.where(gi < E_TOT, jnp.exp(ev - gm), 0.0)
            dvg = dd[pl.ds(g * 16, 16)]
            dv = dvg - nlo
            mine = jnp.logical_and(dv >= 0, dv < HALF)
            dl[pl.ds(g * 16, 16)] = jnp.where(mine, dv, HALF)
            if half == 0:
                bidx[pl.ds(g * 16, 16)] = dvg >> 7
            dm = dvg & (D - 1)

            @plsc.parallel_loop(0, 16, unroll=4)
            def _(k):
                e = g * 16 + k
                kk = jnp.full((16,), k, jnp.int32)
                pvec = jnp.take_along_axis(p16, kk, axis=0)
                for j in range(8):
                    sc[e, pl.ds(j * 16, 16)] = \
                        pvec * xs[e, pl.ds(j * 16, 16)]
                if half == 0:
                    dmk = jnp.take_along_axis(dm, kk, axis=0)
                    for j in range(8):
                        dnbuf[e, pl.ds(j * 16, 16)] = \
                            jnp.where(lanes + (j * 16) == dmk, pvec, 0.0)

            return 0

        lax.fori_loop(0, CHUNK // 16, group_body, 0)

    for half in range(2):
        nlo = half * HALF
        start_scatter(sc0, dl0, ss0)
        start_scatter(sc1, dl1, ss1)
        if half == 0:
            start_dscatter()
        start_idx(0, si0, dd0, de0, ix0)
        start_idx(1, si1, dd1, de1, ix1)
        wait_idx(0, si0, dd0, de0, ix0)
        start_gather(si0, xs0, gx0)

        def do_chunk(ci, si, dd, de, isem, xs, gsem, sc, dl, ssem,
                     nsi, ndd, nde, nisem, nxs, ngsem, last=False):
            wait_gather(si, xs, gsem)
            if not last:
                wait_idx(ci + 1, nsi, ndd, nde, nisem)
                start_gather(nsi, nxs, ngsem)
            wait_scatter(sc, dl, ssem)
            if half == 0:
                wait_dscatter()
            compute(ci, xs, sc, dl, dd, de, half)
            start_idx(ci + 2, si, dd, de, isem)
            start_scatter(sc, dl, ssem)
            if half == 0:
                start_dscatter()

        def pair_body(t, _):
            c0 = 2 * t
            do_chunk(c0, si0, dd0, de0, ix0, xs0, gx0, sc0, dl0, ss0,
                     si1, dd1, de1, ix1, xs1, gx1)
            do_chunk(c0 + 1, si1, dd1, de1, ix1, xs1, gx1, sc1, dl1, ss1,
                     si0, dd0, de0, ix0, xs0, gx0)
            return 0

        lax.fori_loop(0, (NCHUNK - 1) // 2, pair_body, 0)
        do_chunk(NCHUNK - 1, si0, dd0, de0, ix0, xs0, gx0, sc0, dl0, ss0,
                 si1, dd1, de1, ix1, xs1, gx1, last=True)
        wait_idx(NCHUNK - 1, si0, dd0, de0, ix0)
        wait_idx(NCHUNK - 1, si1, dd1, de1, ix1)
        wait_scatter(sc0, dl0, ss0)
        wait_scatter(sc1, dl1, ss1)
        if half == 0:
            wait_dscatter()
        plsc.subcore_barrier()

        def copy_out(rlo, nrows):
            rows = accum.at[pl.ds(rlo, nrows)]

            @pl.when(c == 0)
            def _():
                pltpu.sync_copy(rows, p0_hbm.at[pl.ds(nlo + rlo, nrows)])

            @pl.when(c == 1)
            def _():
                pltpu.sync_copy(rows, p1_hbm.at[pl.ds(nlo + rlo, nrows)])

        for k in range(2):
            copy_out(s * OUT_R + k * CHUNK, CHUNK)
        copy_out(s * OUT_R + 2 * CHUNK, OUT_R - 2 * CHUNK)

        if half == 0:
            plsc.subcore_barrier()
            zero_rows(sc0)
            zero_rows(sc1)
            init_idx(dl0, HALF)
            init_idx(dl1, HALF)
            zero_accum(sc0)
            plsc.subcore_barrier()

    @pl.when(jnp.logical_and(s < DEN_R // 8, c == 0))
    def _():
        pltpu.sync_copy(dacc.at[pl.ds(s * 8, 8)], d0_hbm.at[pl.ds(s * 8, 8)])

    @pl.when(jnp.logical_and(s < DEN_R // 8, c == 1))
    def _():
        pltpu.sync_copy(dacc.at[pl.ds(s * 8, 8)], d1_hbm.at[pl.ds(s * 8, 8)])


def kernel(x, edge_index, params):
    sl = jnp.arange(N, dtype=jnp.int32)
    pad = jnp.zeros((EP - E_TOT,), jnp.int32)
    src = jnp.concatenate([edge_index[0].astype(jnp.int32), sl, pad])
    dst = jnp.concatenate([edge_index[1].astype(jnp.int32), sl, pad])
    xp = jnp.pad(x, ((0, NPAD - N), (0, 0)))

    nl = len(params) // 4
    p0 = p1 = d0 = d1 = bprev = None
    for i in range(nl):
        wl = params[f"Wl{i}"]
        wr = params[f"Wr{i}"]
        att = params[f"att{i}"]
        b = params[f"b{i}"].reshape(1, D)
        if i == 0:
            xl, xr = _mm_first(xp, wl, wr)
        else:
            xl, xr = _combine_mm(p0, p1, d0, d1, bprev, wl, wr)
        e, m = _sc_scores(xl, xr, att, src, dst)
        p0, p1, d0, d1 = _sc_aggregate(xl, src, dst, e, m)
        bprev = b
    return _final_pool(p0, p1, d0, d1, bprev)


# speedup vs baseline: 8.6528x; 1.0190x over previous
import functools
import jax
import jax.numpy as jnp
from jax import lax
from jax.experimental import pallas as pl
from jax.experimental.pallas import tpu as pltpu
from jax.experimental.pallas import tpu_sc as plsc

N = 10000
E_RAW = 320000
E_TOT = E_RAW + N
D = 128
NEG = 0.2
NW = 32
CHUNK = 128
NCHUNK = 81
EPT = NCHUNK * CHUNK
EP = NW * EPT
NPAD = 10240
DEN_R = NPAD // D
ROWB = 1024
NROWB = NPAD // ROWB
DEN_RB = DEN_R // NROWB
HALF = NPAD // 2
ACC_R = HALF + CHUNK
SUB_R = ACC_R // 16
OUT_R = HALF // 16
DEN_H = HALF // D
DACC_R = DEN_H + 8

_mesh = plsc.VectorSubcoreMesh(core_axis_name="c", subcore_axis_name="s")


def _bfly_sum(v, lanes):
    for sh in (8, 4, 2, 1):
        v = v + jnp.take_along_axis(v, jnp.bitwise_xor(lanes, sh), axis=0)
    return v


def _bfly_max(v, lanes):
    for sh in (8, 4, 2, 1):
        v = jnp.maximum(
            v, jnp.take_along_axis(v, jnp.bitwise_xor(lanes, sh), axis=0))
    return v


def _den_column(d_blk):
    r = lax.broadcasted_iota(jnp.int32, (ROWB, DEN_RB), 0)
    k = lax.broadcasted_iota(jnp.int32, (ROWB, DEN_RB), 1)
    sel = (k == (r >> 7)).astype(jnp.float32)
    expanded = jnp.dot(sel, d_blk, preferred_element_type=jnp.float32)
    rr = lax.broadcasted_iota(jnp.int32, (ROWB, D), 0)
    cc = lax.broadcasted_iota(jnp.int32, (ROWB, D), 1)
    mask = (cc == (rr & (D - 1))).astype(jnp.float32)
    return jnp.sum(expanded * mask, axis=1, keepdims=True)


def _mm_first_body(h_ref, wl_ref, wr_ref, xl_ref, xr_ref):
    h = h_ref[...]
    xl_ref[...] = jnp.dot(h, wl_ref[...], preferred_element_type=jnp.float32)
    xr_ref[...] = jnp.dot(h, wr_ref[...], preferred_element_type=jnp.float32)


def _mm_first(h, wl, wr):
    return pl.pallas_call(
        _mm_first_body,
        grid=(NROWB,),
        in_specs=[
            pl.BlockSpec((ROWB, D), lambda i: (i, 0)),
            pl.BlockSpec((D, D), lambda i: (0, 0)),
            pl.BlockSpec((D, D), lambda i: (0, 0)),
        ],
        out_specs=[
            pl.BlockSpec((ROWB, D), lambda i: (i, 0)),
            pl.BlockSpec((ROWB, D), lambda i: (i, 0)),
        ],
        out_shape=[
            jax.ShapeDtypeStruct((NPAD, D), jnp.float32),
            jax.ShapeDtypeStruct((NPAD, D), jnp.float32),
        ],
    )(h, wl, wr)


def _combine_body(p0_ref, p1_ref, d0_ref, d1_ref, b_ref):
    den = _den_column(d0_ref[...] + d1_ref[...])
    return (p0_ref[...] + p1_ref[...]) / (den + 1e-16) + b_ref[...]


def _combine_mm_body(p0_ref, p1_ref, d0_ref, d1_ref, b_ref, wl_ref, wr_ref,
                     xl_ref, xr_ref):
    h = _combine_body(p0_ref, p1_ref, d0_ref, d1_ref, b_ref)
    h = jnp.maximum(h, 0.0)
    xl_ref[...] = jnp.dot(h, wl_ref[...], preferred_element_type=jnp.float32)
    xr_ref[...] = jnp.dot(h, wr_ref[...], preferred_element_type=jnp.float32)


def _combine_mm(p0, p1, d0, d1, b, wl, wr):
    return pl.pallas_call(
        _combine_mm_body,
        grid=(NROWB,),
        in_specs=[
            pl.BlockSpec((ROWB, D), lambda i: (i, 0)),
            pl.BlockSpec((ROWB, D), lambda i: (i, 0)),
            pl.BlockSpec((DEN_RB, D), lambda i: (i, 0)),
            pl.BlockSpec((DEN_RB, D), lambda i: (i, 0)),
            pl.BlockSpec((1, D), lambda i: (0, 0)),
            pl.BlockSpec((D, D), lambda i: (0, 0)),
            pl.BlockSpec((D, D), lambda i: (0, 0)),
        ],
        out_specs=[
            pl.BlockSpec((ROWB, D), lambda i: (i, 0)),
            pl.BlockSpec((ROWB, D), lambda i: (i, 0)),
        ],
        out_shape=[
            jax.ShapeDtypeStruct((NPAD, D), jnp.float32),
            jax.ShapeDtypeStruct((NPAD, D), jnp.float32),
        ],
    )(p0, p1, d0, d1, b, wl, wr)


def _final_body(p0_ref, p1_ref, d0_ref, d1_ref, b_ref, o_ref):
    i = pl.program_id(0)
    h = _combine_body(p0_ref, p1_ref, d0_ref, d1_ref, b_ref)
    gi = i * ROWB + lax.broadcasted_iota(jnp.int32, (ROWB, D), 0)
    h = jnp.where(gi < N, h, 0.0)
    s = jnp.sum(h, axis=0, keepdims=True) * (1.0 / N)

    @pl.when(i == 0)
    def _():
        o_ref[...] = s

    @pl.when(i > 0)
    def _():
        o_ref[...] += s


def _final_pool(p0, p1, d0, d1, b):
    return pl.pallas_call(
        _final_body,
        grid=(NROWB,),
        in_specs=[
            pl.BlockSpec((ROWB, D), lambda i: (i, 0)),
            pl.BlockSpec((ROWB, D), lambda i: (i, 0)),
            pl.BlockSpec((DEN_RB, D), lambda i: (i, 0)),
            pl.BlockSpec((DEN_RB, D), lambda i: (i, 0)),
            pl.BlockSpec((1, D), lambda i: (0, 0)),
        ],
        out_specs=pl.BlockSpec((1, D), lambda i: (0, 0)),
        out_shape=jax.ShapeDtypeStruct((1, D), jnp.float32),
    )(p0, p1, d0, d1, b)


@functools.partial(
    pl.kernel,
    out_type=[
        jax.ShapeDtypeStruct((EP,), jnp.float32),
        jax.ShapeDtypeStruct((NW, 16), jnp.float32),
    ],
    mesh=_mesh,
    scratch_types=[
        pltpu.VMEM((CHUNK,), jnp.int32),
        pltpu.VMEM((CHUNK,), jnp.int32),
        pltpu.VMEM((CHUNK,), jnp.int32),
        pltpu.VMEM((CHUNK,), jnp.int32),
        pltpu.VMEM((CHUNK, D), jnp.float32),
        pltpu.VMEM((CHUNK, D), jnp.float32),
        pltpu.VMEM((CHUNK, D), jnp.float32),
        pltpu.VMEM((CHUNK, D), jnp.float32),
        pltpu.VMEM((CHUNK,), jnp.float32),
        pltpu.VMEM((D,), jnp.float32),
        pltpu.VMEM((16,), jnp.float32),
        pltpu.SemaphoreType.DMA,
        pltpu.SemaphoreType.DMA,
        pltpu.SemaphoreType.DMA,
        pltpu.SemaphoreType.DMA,
    ],
)
def _sc_scores(xl_hbm, xr_hbm, att_hbm, src_hbm, dst_hbm, e_hbm, m_hbm,
               si0, si1, dd0, dd1, xs0, xs1, xr0, xr1, ebuf, attv, mbuf,
               gp0, gp1, ix0, ix1):
    c = lax.axis_index("c")
    s = lax.axis_index("s")
    wid = s * 2 + c
    base = wid * EPT
    pltpu.sync_copy(att_hbm, attv)
    lanes = lax.iota(jnp.int32, 16)

    def start_idx(ci, si, dd, sem):
        cc = jnp.minimum(ci, NCHUNK - 1)
        off = base + cc * CHUNK
        pltpu.async_copy(src_hbm.at[pl.ds(off, CHUNK)], si, sem)
        pltpu.async_copy(dst_hbm.at[pl.ds(off, CHUNK)], dd, sem)

    def wait_idx(ci, si, dd, sem):
        cc = jnp.minimum(ci, NCHUNK - 1)
        off = base + cc * CHUNK
        pltpu.make_async_copy(src_hbm.at[pl.ds(off, CHUNK)], si, sem).wait()
        pltpu.make_async_copy(dst_hbm.at[pl.ds(off, CHUNK)], dd, sem).wait()

    def start_gather(si, dd, xs, xr, sem):
        pltpu.async_copy(xl_hbm.at[si], xs, sem)
        pltpu.async_copy(xr_hbm.at[dd], xr, sem)

    def wait_gather(si, dd, xs, xr, sem):
        pltpu.make_async_copy(xl_hbm.at[si], xs, sem).wait()
        pltpu.make_async_copy(xr_hbm.at[dd], xr, sem).wait()

    def compute(ci, xs, xr, m16):
        def group_body(g, m16i):
            @plsc.parallel_loop(0, 16, unroll=8,
                                carry=jnp.zeros((16,), jnp.float32))
            def e16(k, e16i):
                e = g * 16 + k
                acc = jnp.zeros((16,), jnp.float32)
                for j in range(8):
                    a = xs[e, pl.ds(j * 16, 16)] + xr[e, pl.ds(j * 16, 16)]
                    lk = jnp.maximum(a, NEG * a)
                    acc = acc + attv[pl.ds(j * 16, 16)] * lk
                return jnp.where(lanes == k, _bfly_sum(acc, lanes), e16i)

            ebuf[pl.ds(g * 16, 16)] = e16
            return jnp.maximum(m16i, e16)

        m16 = lax.fori_loop(0, CHUNK // 16, group_body, m16)
        pltpu.sync_copy(ebuf, e_hbm.at[pl.ds(base + ci * CHUNK, CHUNK)])
        return m16

    def do_chunk(ci, si, dd, isem, xs, xr, gsem,
                 nsi, ndd, nisem, nxs, nxr, ngsem, m16, last=False):
        wait_gather(si, dd, xs, xr, gsem)
        if not last:
            wait_idx(ci + 1, nsi, ndd, nisem)
            start_gather(nsi, ndd, nxs, nxr, ngsem)
        m16 = compute(ci, xs, xr, m16)
        start_idx(ci + 2, si, dd, isem)
        return m16

    start_idx(0, si0, dd0, ix0)
    start_idx(1, si1, dd1, ix1)
    wait_idx(0, si0, dd0, ix0)
    start_gather(si0, dd0, xs0, xr0, gp0)

    def pair_body(t, m16):
        c0 = 2 * t
        m16 = do_chunk(c0, si0, dd0, ix0, xs0, xr0, gp0,
                       si1, dd1, ix1, xs1, xr1, gp1, m16)
        m16 = do_chunk(c0 + 1, si1, dd1, ix1, xs1, xr1, gp1,
                       si0, dd0, ix0, xs0, xr0, gp0, m16)
        return m16

    m16 = lax.fori_loop(0, (NCHUNK - 1) // 2, pair_body,
                        jnp.full((16,), -3e38, jnp.float32))
    m16 = do_chunk(NCHUNK - 1, si0, dd0, ix0, xs0, xr0, gp0,
                   si1, dd1, ix1, xs1, xr1, gp1, m16, last=True)
    wait_idx(NCHUNK - 1, si0, dd0, ix0)
    wait_idx(NCHUNK - 1, si1, dd1, ix1)
    mbuf[...] = m16
    pltpu.sync_copy(mbuf, m_hbm.at[wid])


@functools.partial(
    pl.kernel,
    out_type=[
        jax.ShapeDtypeStruct((NPAD, D), jnp.float32),
        jax.ShapeDtypeStruct((NPAD, D), jnp.float32),
        jax.ShapeDtypeStruct((DEN_R, D), jnp.float32),
        jax.ShapeDtypeStruct((DEN_R, D), jnp.float32),
    ],
    mesh=_mesh,
    scratch_types=[
        pltpu.VMEM((CHUNK,), jnp.int32),
        pltpu.VMEM((CHUNK,), jnp.int32),
        pltpu.VMEM((CHUNK,), jnp.int32),
        pltpu.VMEM((CHUNK,), jnp.int32),
        pltpu.VMEM((CHUNK,), jnp.float32),
        pltpu.VMEM((CHUNK,), jnp.float32),
        pltpu.VMEM((CHUNK,), jnp.int32),
        pltpu.VMEM((CHUNK,), jnp.int32),
        pltpu.VMEM((CHUNK,), jnp.int32),
        pltpu.VMEM((CHUNK, D), jnp.float32),
        pltpu.VMEM((CHUNK, D), jnp.float32),
        pltpu.VMEM((CHUNK, D), jnp.float32),
        pltpu.VMEM((CHUNK, D), jnp.float32),
        pltpu.VMEM((CHUNK, D), jnp.float32),
        pltpu.VMEM((NW, 16), jnp.float32),
        pltpu.VMEM_SHARED((ACC_R, D), jnp.float32),
        pltpu.VMEM_SHARED((DEN_R, D), jnp.float32),
        pltpu.SemaphoreType.DMA,
        pltpu.SemaphoreType.DMA,
        pltpu.SemaphoreType.DMA,
        pltpu.SemaphoreType.DMA,
        pltpu.SemaphoreType.DMA,
        pltpu.SemaphoreType.DMA,
        pltpu.SemaphoreType.DMA,
    ],
)
def _sc_aggregate(xl_hbm, src_hbm, dst_hbm, e_hbm, m_hbm,
                  p0_hbm, p1_hbm, d0_hbm, d1_hbm,
                  si0, si1, dd0, dd1, de0, de1, dl0, dl1, bidx,
                  xs0, xs1, sc0, sc1, dnbuf, mall, accum, dacc,
                  gx0, gx1, ss0, ss1, sd, ix0, ix1):
    c = lax.axis_index("c")
    s = lax.axis_index("s")
    wid = s * 2 + c
    base = wid * EPT
    lanes = lax.iota(jnp.int32, 16)

    pltpu.sync_copy(m_hbm, mall)
    gm16 = mall[0, :]
    for i in range(1, NW):
        gm16 = jnp.maximum(gm16, mall[i, :])
    gm = _bfly_max(gm16, lanes)

    def zero_accum(zsrc):
        for k in range(2):
            pltpu.sync_copy(zsrc,
                            accum.at[pl.ds(s * SUB_R + k * CHUNK, CHUNK)])
        pltpu.sync_copy(
            zsrc.at[pl.ds(0, SUB_R - 2 * CHUNK)],
            accum.at[pl.ds(s * SUB_R + 2 * CHUNK, SUB_R - 2 * CHUNK)])

    def start_idx(ci, si, dd, de, sem):
        cc = jnp.minimum(ci, NCHUNK - 1)
        off = base + cc * CHUNK
        pltpu.async_copy(src_hbm.at[pl.ds(off, CHUNK)], si, sem)
        pltpu.async_copy(dst_hbm.at[pl.ds(off, CHUNK)], dd, sem)
        pltpu.async_copy(e_hbm.at[pl.ds(off, CHUNK)], de, sem)

    def wait_idx(ci, si, dd, de, sem):
        cc = jnp.minimum(ci, NCHUNK - 1)
        off = base + cc * CHUNK
        pltpu.make_async_copy(src_hbm.at[pl.ds(off, CHUNK)], si, sem).wait()
        pltpu.make_async_copy(dst_hbm.at[pl.ds(off, CHUNK)], dd, sem).wait()
        pltpu.make_async_copy(e_hbm.at[pl.ds(off, CHUNK)], de, sem).wait()

    def start_gather(si, xs, sem):
        pltpu.async_copy(xl_hbm.at[si], xs, sem)

    def wait_gather(si, xs, sem):
        pltpu.make_async_copy(xl_hbm.at[si], xs, sem).wait()

    def start_scatter(sc, dl, sem):
        pltpu.async_copy(sc, accum.at[dl], sem, add=True)

    def wait_scatter(sc, dl, sem):
        pltpu.make_async_copy(sc, accum.at[dl], sem).wait()

    def start_dscatter():
        pltpu.async_copy(dnbuf, dacc.at[bidx], sd, add=True)

    def wait_dscatter():
        pltpu.make_async_copy(dnbuf, dacc.at[bidx], sd).wait()

    def init_idx(buf, val):
        def ib(g, _):
            buf[pl.ds(g * 16, 16)] = jnp.full((16,), val, jnp.int32)
            return 0
        lax.fori_loop(0, CHUNK // 16, ib, 0)

    def zero_rows(buf):
        def zr(e, _):
            for j in range(D // 16):
                buf[e, pl.ds(j * 16, 16)] = jnp.zeros((16,), jnp.float32)
            return 0
        lax.fori_loop(0, CHUNK, zr, 0)

    zero_rows(sc0)
    zero_rows(sc1)
    zero_rows(dnbuf)
    init_idx(dl0, HALF)
    init_idx(dl1, HALF)
    init_idx(bidx, 0)
    zero_accum(sc0)

    @pl.when(s == 0)
    def _():
        pltpu.sync_copy(sc0.at[pl.ds(0, DEN_R)], dacc)

    plsc.subcore_barrier()

    def compute(ci, xs, sc, dl, dd, de, half):
        lo = ci * CHUNK
        nlo = half * HALF

        def group_body(g, _):
            ev = de[pl.ds(g * 16, 16)]
            gi = base + lo + g * 16 + lanes
            p16 = jnp.where(gi < E_TOT, jnp.exp(ev - gm), 0.0)
            dvg = dd[pl.ds(g * 16, 16)]
            dv = dvg - nlo
            mine = jnp.logical_and(dv >= 0, dv < HALF)
            dl[pl.ds(g * 16, 16)] = jnp.where(mine, dv, HALF)
            if half == 0:
                bidx[pl.ds(g * 16, 16)] = dvg >> 7
            dm = dvg & (D - 1)

            @plsc.parallel_loop(0, 16, unroll=8)
            def _(k):
                e = g * 16 + k
                kk = jnp.full((16,), k, jnp.int32)
                pvec = jnp.take_along_axis(p16, kk, axis=0)
                for j in range(8):
                    sc[e, pl.ds(j * 16, 16)] = \
                        pvec * xs[e, pl.ds(j * 16, 16)]
                if half == 0:
                    dmk = jnp.take_along_axis(dm, kk, axis=0)
                    for j in range(8):
                        dnbuf[e, pl.ds(j * 16, 16)] = \
                            jnp.where(lanes + (j * 16) == dmk, pvec, 0.0)

            return 0

        lax.fori_loop(0, CHUNK // 16, group_body, 0)

    for half in range(2):
        nlo = half * HALF
        start_scatter(sc0, dl0, ss0)
        start_scatter(sc1, dl1, ss1)
        if half == 0:
            start_dscatter()
        start_idx(0, si0, dd0, de0, ix0)
        start_idx(1, si1, dd1, de1, ix1)
        wait_idx(0, si0, dd0, de0, ix0)
        start_gather(si0, xs0, gx0)

        def do_chunk(ci, si, dd, de, isem, xs, gsem, sc, dl, ssem,
                     nsi, ndd, nde, nisem, nxs, ngsem, last=False):
            wait_gather(si, xs, gsem)
            if not last:
                wait_idx(ci + 1, nsi, ndd, nde, nisem)
                start_gather(nsi, nxs, ngsem)
            wait_scatter(sc, dl, ssem)
            if half == 0:
                wait_dscatter()
            compute(ci, xs, sc, dl, dd, de, half)
            start_idx(ci + 2, si, dd, de, isem)
            start_scatter(sc, dl, ssem)
            if half == 0:
                start_dscatter()

        def pair_body(t, _):
            c0 = 2 * t
            do_chunk(c0, si0, dd0, de0, ix0, xs0, gx0, sc0, dl0, ss0,
                     si1, dd1, de1, ix1, xs1, gx1)
            do_chunk(c0 + 1, si1, dd1, de1, ix1, xs1, gx1, sc1, dl1, ss1,
                     si0, dd0, de0, ix0, xs0, gx0)
            return 0

        lax.fori_loop(0, (NCHUNK - 1) // 2, pair_body, 0)
        do_chunk(NCHUNK - 1, si0, dd0, de0, ix0, xs0, gx0, sc0, dl0, ss0,
                 si1, dd1, de1, ix1, xs1, gx1, last=True)
        wait_idx(NCHUNK - 1, si0, dd0, de0, ix0)
        wait_idx(NCHUNK - 1, si1, dd1, de1, ix1)
        wait_scatter(sc0, dl0, ss0)
        wait_scatter(sc1, dl1, ss1)
        if half == 0:
            wait_dscatter()
        plsc.subcore_barrier()

        def copy_out(rlo, nrows):
            rows = accum.at[pl.ds(rlo, nrows)]

            @pl.when(c == 0)
            def _():
                pltpu.sync_copy(rows, p0_hbm.at[pl.ds(nlo + rlo, nrows)])

            @pl.when(c == 1)
            def _():
                pltpu.sync_copy(rows, p1_hbm.at[pl.ds(nlo + rlo, nrows)])

        for k in range(2):
            copy_out(s * OUT_R + k * CHUNK, CHUNK)
        copy_out(s * OUT_R + 2 * CHUNK, OUT_R - 2 * CHUNK)

        if half == 0:
            plsc.subcore_barrier()
            zero_rows(sc0)
            zero_rows(sc1)
            init_idx(dl0, HALF)
            init_idx(dl1, HALF)
            zero_accum(sc0)
            plsc.subcore_barrier()

    @pl.when(jnp.logical_and(s < DEN_R // 8, c == 0))
    def _():
        pltpu.sync_copy(dacc.at[pl.ds(s * 8, 8)], d0_hbm.at[pl.ds(s * 8, 8)])

    @pl.when(jnp.logical_and(s < DEN_R // 8, c == 1))
    def _():
        pltpu.sync_copy(dacc.at[pl.ds(s * 8, 8)], d1_hbm.at[pl.ds(s * 8, 8)])


def kernel(x, edge_index, params):
    sl = jnp.arange(N, dtype=jnp.int32)
    pad = jnp.zeros((EP - E_TOT,), jnp.int32)
    src = jnp.concatenate([edge_index[0].astype(jnp.int32), sl, pad])
    dst = jnp.concatenate([edge_index[1].astype(jnp.int32), sl, pad])
    xp = jnp.pad(x, ((0, NPAD - N), (0, 0)))

    nl = len(params) // 4
    p0 = p1 = d0 = d1 = bprev = None
    for i in range(nl):
        wl = params[f"Wl{i}"]
        wr = params[f"Wr{i}"]
        att = params[f"att{i}"]
        b = params[f"b{i}"].reshape(1, D)
        if i == 0:
            xl, xr = _mm_first(xp, wl, wr)
        else:
            xl, xr = _combine_mm(p0, p1, d0, d1, bprev, wl, wr)
        e, m = _sc_scores(xl, xr, att, src, dst)
        p0, p1, d0, d1 = _sc_aggregate(xl, src, dst, e, m)
        bprev = b
    return _final_pool(p0, p1, d0, d1, bprev)


# speedup vs baseline: 9.1596x; 1.0586x over previous
import functools
import jax
import jax.numpy as jnp
from jax import lax
from jax.experimental import pallas as pl
from jax.experimental.pallas import tpu as pltpu
from jax.experimental.pallas import tpu_sc as plsc

N = 10000
E_RAW = 320000
E_TOT = E_RAW + N
D = 128
NEG = 0.2
NW = 32
CHUNK = 128
NCHUNK = 81
EPT = NCHUNK * CHUNK
EP = NW * EPT
NPAD = 10240
DEN_R = NPAD // D
ROWB = 1024
NROWB = NPAD // ROWB
DEN_RB = DEN_R // NROWB
HALF = NPAD // 2
ACC_R = HALF + CHUNK
SUB_R = ACC_R // 16
OUT_R = HALF // 16
DEN_H = HALF // D
DACC_R = DEN_H + 8

_mesh = plsc.VectorSubcoreMesh(core_axis_name="c", subcore_axis_name="s")


def _bfly_sum(v, lanes):
    for sh in (8, 4, 2, 1):
        v = v + jnp.take_along_axis(v, jnp.bitwise_xor(lanes, sh), axis=0)
    return v


def _bfly_max(v, lanes):
    for sh in (8, 4, 2, 1):
        v = jnp.maximum(
            v, jnp.take_along_axis(v, jnp.bitwise_xor(lanes, sh), axis=0))
    return v


def _den_column(d_blk):
    r = lax.broadcasted_iota(jnp.int32, (ROWB, DEN_RB), 0)
    k = lax.broadcasted_iota(jnp.int32, (ROWB, DEN_RB), 1)
    sel = (k == (r >> 7)).astype(jnp.float32)
    expanded = jnp.dot(sel, d_blk, preferred_element_type=jnp.float32)
    rr = lax.broadcasted_iota(jnp.int32, (ROWB, D), 0)
    cc = lax.broadcasted_iota(jnp.int32, (ROWB, D), 1)
    mask = (cc == (rr & (D - 1))).astype(jnp.float32)
    return jnp.sum(expanded * mask, axis=1, keepdims=True)


def _mm_first_body(h_ref, wl_ref, wr_ref, xl_ref, xr_ref):
    h = h_ref[...]
    xl_ref[...] = jnp.dot(h, wl_ref[...], preferred_element_type=jnp.float32)
    xr_ref[...] = jnp.dot(h, wr_ref[...], preferred_element_type=jnp.float32)


def _mm_first(h, wl, wr):
    return pl.pallas_call(
        _mm_first_body,
        grid=(NROWB,),
        in_specs=[
            pl.BlockSpec((ROWB, D), lambda i: (i, 0)),
            pl.BlockSpec((D, D), lambda i: (0, 0)),
            pl.BlockSpec((D, D), lambda i: (0, 0)),
        ],
        out_specs=[
            pl.BlockSpec((ROWB, D), lambda i: (i, 0)),
            pl.BlockSpec((ROWB, D), lambda i: (i, 0)),
        ],
        out_shape=[
            jax.ShapeDtypeStruct((NPAD, D), jnp.float32),
            jax.ShapeDtypeStruct((NPAD, D), jnp.float32),
        ],
    )(h, wl, wr)


def _combine_body(p0_ref, p1_ref, d0_ref, d1_ref, b_ref):
    den = _den_column(d0_ref[...] + d1_ref[...])
    return (p0_ref[...] + p1_ref[...]) / (den + 1e-16) + b_ref[...]


def _combine_mm_body(p0_ref, p1_ref, d0_ref, d1_ref, b_ref, wl_ref, wr_ref,
                     xl_ref, xr_ref):
    h = _combine_body(p0_ref, p1_ref, d0_ref, d1_ref, b_ref)
    h = jnp.maximum(h, 0.0)
    xl_ref[...] = jnp.dot(h, wl_ref[...], preferred_element_type=jnp.float32)
    xr_ref[...] = jnp.dot(h, wr_ref[...], preferred_element_type=jnp.float32)


def _combine_mm(p0, p1, d0, d1, b, wl, wr):
    return pl.pallas_call(
        _combine_mm_body,
        grid=(NROWB,),
        in_specs=[
            pl.BlockSpec((ROWB, D), lambda i: (i, 0)),
            pl.BlockSpec((ROWB, D), lambda i: (i, 0)),
            pl.BlockSpec((DEN_RB, D), lambda i: (i, 0)),
            pl.BlockSpec((DEN_RB, D), lambda i: (i, 0)),
            pl.BlockSpec((1, D), lambda i: (0, 0)),
            pl.BlockSpec((D, D), lambda i: (0, 0)),
            pl.BlockSpec((D, D), lambda i: (0, 0)),
        ],
        out_specs=[
            pl.BlockSpec((ROWB, D), lambda i: (i, 0)),
            pl.BlockSpec((ROWB, D), lambda i: (i, 0)),
        ],
        out_shape=[
            jax.ShapeDtypeStruct((NPAD, D), jnp.float32),
            jax.ShapeDtypeStruct((NPAD, D), jnp.float32),
        ],
    )(p0, p1, d0, d1, b, wl, wr)


def _final_body(p0_ref, p1_ref, d0_ref, d1_ref, b_ref, o_ref):
    i = pl.program_id(0)
    h = _combine_body(p0_ref, p1_ref, d0_ref, d1_ref, b_ref)
    gi = i * ROWB + lax.broadcasted_iota(jnp.int32, (ROWB, D), 0)
    h = jnp.where(gi < N, h, 0.0)
    s = jnp.sum(h, axis=0, keepdims=True) * (1.0 / N)

    @pl.when(i == 0)
    def _():
        o_ref[...] = s

    @pl.when(i > 0)
    def _():
        o_ref[...] += s


def _final_pool(p0, p1, d0, d1, b):
    return pl.pallas_call(
        _final_body,
        grid=(NROWB,),
        in_specs=[
            pl.BlockSpec((ROWB, D), lambda i: (i, 0)),
            pl.BlockSpec((ROWB, D), lambda i: (i, 0)),
            pl.BlockSpec((DEN_RB, D), lambda i: (i, 0)),
            pl.BlockSpec((DEN_RB, D), lambda i: (i, 0)),
            pl.BlockSpec((1, D), lambda i: (0, 0)),
        ],
        out_specs=pl.BlockSpec((1, D), lambda i: (0, 0)),
        out_shape=jax.ShapeDtypeStruct((1, D), jnp.float32),
    )(p0, p1, d0, d1, b)


@functools.partial(
    pl.kernel,
    out_type=[
        jax.ShapeDtypeStruct((EP,), jnp.float32),
        jax.ShapeDtypeStruct((NW, 16), jnp.float32),
    ],
    mesh=_mesh,
    scratch_types=[
        pltpu.VMEM((CHUNK,), jnp.int32),
        pltpu.VMEM((CHUNK,), jnp.int32),
        pltpu.VMEM((CHUNK,), jnp.int32),
        pltpu.VMEM((CHUNK,), jnp.int32),
        pltpu.VMEM((CHUNK, D), jnp.float32),
        pltpu.VMEM((CHUNK, D), jnp.float32),
        pltpu.VMEM((CHUNK, D), jnp.float32),
        pltpu.VMEM((CHUNK, D), jnp.float32),
        pltpu.VMEM((CHUNK,), jnp.float32),
        pltpu.VMEM((D,), jnp.float32),
        pltpu.VMEM((16,), jnp.float32),
        pltpu.SemaphoreType.DMA,
        pltpu.SemaphoreType.DMA,
        pltpu.SemaphoreType.DMA,
        pltpu.SemaphoreType.DMA,
    ],
)
def _sc_scores(xl_hbm, xr_hbm, att_hbm, src_hbm, dst_hbm, e_hbm, m_hbm,
               si0, si1, dd0, dd1, xs0, xs1, xr0, xr1, ebuf, attv, mbuf,
               gp0, gp1, ix0, ix1):
    c = lax.axis_index("c")
    s = lax.axis_index("s")
    wid = s * 2 + c
    base = wid * EPT
    pltpu.sync_copy(att_hbm, attv)
    lanes = lax.iota(jnp.int32, 16)

    def start_idx(ci, si, dd, sem):
        cc = jnp.minimum(ci, NCHUNK - 1)
        off = base + cc * CHUNK
        pltpu.async_copy(src_hbm.at[pl.ds(off, CHUNK)], si, sem)
        pltpu.async_copy(dst_hbm.at[pl.ds(off, CHUNK)], dd, sem)

    def wait_idx(ci, si, dd, sem):
        cc = jnp.minimum(ci, NCHUNK - 1)
        off = base + cc * CHUNK
        pltpu.make_async_copy(src_hbm.at[pl.ds(off, CHUNK)], si, sem).wait()
        pltpu.make_async_copy(dst_hbm.at[pl.ds(off, CHUNK)], dd, sem).wait()

    def start_gather(si, dd, xs, xr, sem):
        pltpu.async_copy(xl_hbm.at[si], xs, sem)
        pltpu.async_copy(xr_hbm.at[dd], xr, sem)

    def wait_gather(si, dd, xs, xr, sem):
        pltpu.make_async_copy(xl_hbm.at[si], xs, sem).wait()
        pltpu.make_async_copy(xr_hbm.at[dd], xr, sem).wait()

    def compute(ci, xs, xr, m16):
        def group_body(g, m16i):
            @plsc.parallel_loop(0, 16, unroll=8,
                                carry=jnp.zeros((16,), jnp.float32))
            def e16(k, e16i):
                e = g * 16 + k
                acc = jnp.zeros((16,), jnp.float32)
                for j in range(8):
                    a = xs[e, pl.ds(j * 16, 16)] + xr[e, pl.ds(j * 16, 16)]
                    lk = jnp.maximum(a, NEG * a)
                    acc = acc + attv[pl.ds(j * 16, 16)] * lk
                return jnp.where(lanes == k, _bfly_sum(acc, lanes), e16i)

            ebuf[pl.ds(g * 16, 16)] = e16
            return jnp.maximum(m16i, e16)

        m16 = lax.fori_loop(0, CHUNK // 16, group_body, m16)
        pltpu.sync_copy(ebuf, e_hbm.at[pl.ds(base + ci * CHUNK, CHUNK)])
        return m16

    def do_chunk(ci, si, dd, isem, xs, xr, gsem,
                 nsi, ndd, nisem, nxs, nxr, ngsem, m16, last=False):
        wait_gather(si, dd, xs, xr, gsem)
        if not last:
            wait_idx(ci + 1, nsi, ndd, nisem)
            start_gather(nsi, ndd, nxs, nxr, ngsem)
        m16 = compute(ci, xs, xr, m16)
        start_idx(ci + 2, si, dd, isem)
        return m16

    start_idx(0, si0, dd0, ix0)
    start_idx(1, si1, dd1, ix1)
    wait_idx(0, si0, dd0, ix0)
    start_gather(si0, dd0, xs0, xr0, gp0)

    def pair_body(t, m16):
        c0 = 2 * t
        m16 = do_chunk(c0, si0, dd0, ix0, xs0, xr0, gp0,
                       si1, dd1, ix1, xs1, xr1, gp1, m16)
        m16 = do_chunk(c0 + 1, si1, dd1, ix1, xs1, xr1, gp1,
                       si0, dd0, ix0, xs0, xr0, gp0, m16)
        return m16

    m16 = lax.fori_loop(0, (NCHUNK - 1) // 2, pair_body,
                        jnp.full((16,), -3e38, jnp.float32))
    m16 = do_chunk(NCHUNK - 1, si0, dd0, ix0, xs0, xr0, gp0,
                   si1, dd1, ix1, xs1, xr1, gp1, m16, last=True)
    wait_idx(NCHUNK - 1, si0, dd0, ix0)
    wait_idx(NCHUNK - 1, si1, dd1, ix1)
    mbuf[...] = m16
    pltpu.sync_copy(mbuf, m_hbm.at[wid])


@functools.partial(
    pl.kernel,
    out_type=[
        jax.ShapeDtypeStruct((NPAD, D), jnp.float32),
        jax.ShapeDtypeStruct((NPAD, D), jnp.float32),
        jax.ShapeDtypeStruct((DEN_R, D), jnp.float32),
        jax.ShapeDtypeStruct((DEN_R, D), jnp.float32),
        jax.ShapeDtypeStruct((EP + CHUNK, D), jnp.float32),
    ],
    mesh=_mesh,
    scratch_types=[
        pltpu.VMEM((CHUNK,), jnp.int32),
        pltpu.VMEM((CHUNK,), jnp.int32),
        pltpu.VMEM((CHUNK,), jnp.int32),
        pltpu.VMEM((CHUNK,), jnp.int32),
        pltpu.VMEM((CHUNK,), jnp.float32),
        pltpu.VMEM((CHUNK,), jnp.float32),
        pltpu.VMEM((CHUNK,), jnp.int32),
        pltpu.VMEM((CHUNK,), jnp.int32),
        pltpu.VMEM((CHUNK,), jnp.int32),
        pltpu.VMEM((CHUNK, D), jnp.float32),
        pltpu.VMEM((CHUNK, D), jnp.float32),
        pltpu.VMEM((CHUNK, D), jnp.float32),
        pltpu.VMEM((CHUNK, D), jnp.float32),
        pltpu.VMEM((CHUNK, D), jnp.float32),
        pltpu.VMEM((NW, 16), jnp.float32),
        pltpu.VMEM_SHARED((ACC_R, D), jnp.float32),
        pltpu.VMEM_SHARED((DEN_R, D), jnp.float32),
        pltpu.SemaphoreType.DMA,
        pltpu.SemaphoreType.DMA,
        pltpu.SemaphoreType.DMA,
        pltpu.SemaphoreType.DMA,
        pltpu.SemaphoreType.DMA,
        pltpu.SemaphoreType.DMA,
        pltpu.SemaphoreType.DMA,
        pltpu.SemaphoreType.DMA,
        pltpu.SemaphoreType.DMA,
    ],
)
def _sc_aggregate(xl_hbm, src_hbm, dst_hbm, e_hbm, m_hbm,
                  p0_hbm, p1_hbm, d0_hbm, d1_hbm, sr_hbm,
                  si0, si1, dd0, dd1, de0, de1, dl0, dl1, bidx,
                  xs0, xs1, sc0, sc1, dnbuf, mall, accum, dacc,
                  gx0, gx1, ss0, ss1, sd, ix0, ix1, sw0, sw1):
    c = lax.axis_index("c")
    s = lax.axis_index("s")
    wid = s * 2 + c
    base = wid * EPT
    lanes = lax.iota(jnp.int32, 16)

    pltpu.sync_copy(m_hbm, mall)
    gm16 = mall[0, :]
    for i in range(1, NW):
        gm16 = jnp.maximum(gm16, mall[i, :])
    gm = _bfly_max(gm16, lanes)

    def zero_accum(zsrc):
        for k in range(2):
            pltpu.sync_copy(zsrc,
                            accum.at[pl.ds(s * SUB_R + k * CHUNK, CHUNK)])
        pltpu.sync_copy(
            zsrc.at[pl.ds(0, SUB_R - 2 * CHUNK)],
            accum.at[pl.ds(s * SUB_R + 2 * CHUNK, SUB_R - 2 * CHUNK)])

    def start_idx(ci, si, dd, de, sem):
        cc = jnp.minimum(ci, NCHUNK - 1)
        off = base + cc * CHUNK
        pltpu.async_copy(src_hbm.at[pl.ds(off, CHUNK)], si, sem)
        pltpu.async_copy(dst_hbm.at[pl.ds(off, CHUNK)], dd, sem)
        pltpu.async_copy(e_hbm.at[pl.ds(off, CHUNK)], de, sem)

    def wait_idx(ci, si, dd, de, sem):
        cc = jnp.minimum(ci, NCHUNK - 1)
        off = base + cc * CHUNK
        pltpu.make_async_copy(src_hbm.at[pl.ds(off, CHUNK)], si, sem).wait()
        pltpu.make_async_copy(dst_hbm.at[pl.ds(off, CHUNK)], dd, sem).wait()
        pltpu.make_async_copy(e_hbm.at[pl.ds(off, CHUNK)], de, sem).wait()

    def start_gather(si, xs, sem):
        pltpu.async_copy(xl_hbm.at[si], xs, sem)

    def wait_gather(si, xs, sem):
        pltpu.make_async_copy(xl_hbm.at[si], xs, sem).wait()

    def start_scatter(sc, dl, sem):
        pltpu.async_copy(sc, accum.at[dl], sem, add=True)

    def wait_scatter(sc, dl, sem):
        pltpu.make_async_copy(sc, accum.at[dl], sem).wait()

    def start_dscatter():
        pltpu.async_copy(dnbuf, dacc.at[bidx], sd, add=True)

    def wait_dscatter():
        pltpu.make_async_copy(dnbuf, dacc.at[bidx], sd).wait()

    def init_idx(buf, val):
        def ib(g, _):
            buf[pl.ds(g * 16, 16)] = jnp.full((16,), val, jnp.int32)
            return 0
        lax.fori_loop(0, CHUNK // 16, ib, 0)

    def zero_rows(buf):
        def zr(e, _):
            for j in range(D // 16):
                buf[e, pl.ds(j * 16, 16)] = jnp.zeros((16,), jnp.float32)
            return 0
        lax.fori_loop(0, CHUNK, zr, 0)

    zero_rows(sc0)
    zero_rows(sc1)
    zero_rows(dnbuf)
    init_idx(dl0, HALF)
    init_idx(dl1, HALF)
    init_idx(bidx, 0)
    zero_accum(sc0)

    @pl.when(s == 0)
    def _():
        pltpu.sync_copy(sc0.at[pl.ds(0, DEN_R)], dacc)

    plsc.subcore_barrier()

    def compute(ci, xs, sc, dl, dd, de, half):
        lo = ci * CHUNK
        nlo = half * HALF

        def group_body(g, _):
            ev = de[pl.ds(g * 16, 16)]
            gi = base + lo + g * 16 + lanes
            p16 = jnp.where(gi < E_TOT, jnp.exp(ev - gm), 0.0)
            dvg = dd[pl.ds(g * 16, 16)]
            dv = dvg - nlo
            mine = jnp.logical_and(dv >= 0, dv < HALF)
            dl[pl.ds(g * 16, 16)] = jnp.where(mine, dv, HALF)
            if half == 0:
                bidx[pl.ds(g * 16, 16)] = dvg >> 7
            dm = dvg & (D - 1)

            @plsc.parallel_loop(0, 16, unroll=8)
            def _(k):
                e = g * 16 + k
                kk = jnp.full((16,), k, jnp.int32)
                pvec = jnp.take_along_axis(p16, kk, axis=0)
                for j in range(8):
                    sc[e, pl.ds(j * 16, 16)] = \
                        pvec * xs[e, pl.ds(j * 16, 16)]
                if half == 0:
                    dmk = jnp.take_along_axis(dm, kk, axis=0)
                    for j in range(8):
                        dnbuf[e, pl.ds(j * 16, 16)] = \
                            jnp.where(lanes + (j * 16) == dmk, pvec, 0.0)

            return 0

        lax.fori_loop(0, CHUNK // 16, group_body, 0)

    def sr_start(ci, sc, sem):
        pltpu.async_copy(sc, sr_hbm.at[pl.ds(base + ci * CHUNK, CHUNK)], sem)

    def sr_wait(ci, sc, sem):
        pltpu.make_async_copy(
            sc, sr_hbm.at[pl.ds(base + ci * CHUNK, CHUNK)], sem).wait()

    def sr_prime(sc, sem):
        pltpu.async_copy(sc, sr_hbm.at[pl.ds(EP, CHUNK)], sem)

    def copy_out(nlo, rlo, nrows):
        rows = accum.at[pl.ds(rlo, nrows)]

        @pl.when(c == 0)
        def _():
            pltpu.sync_copy(rows, p0_hbm.at[pl.ds(nlo + rlo, nrows)])

        @pl.when(c == 1)
        def _():
            pltpu.sync_copy(rows, p1_hbm.at[pl.ds(nlo + rlo, nrows)])

    def copy_out_all(nlo):
        for k in range(2):
            copy_out(nlo, s * OUT_R + k * CHUNK, CHUNK)
        copy_out(nlo, s * OUT_R + 2 * CHUNK, OUT_R - 2 * CHUNK)

    start_scatter(sc0, dl0, ss0)
    start_scatter(sc1, dl1, ss1)
    start_dscatter()
    sr_prime(sc0, sw0)
    sr_prime(sc1, sw1)
    start_idx(0, si0, dd0, de0, ix0)
    start_idx(1, si1, dd1, de1, ix1)
    wait_idx(0, si0, dd0, de0, ix0)
    start_gather(si0, xs0, gx0)

    def do_chunk0(ci, si, dd, de, isem, xs, gsem, sc, dl, ssem, swsem,
                  nsi, ndd, nde, nisem, nxs, ngsem, last=False):
        wait_gather(si, xs, gsem)
        if not last:
            wait_idx(ci + 1, nsi, ndd, nde, nisem)
            start_gather(nsi, nxs, ngsem)
        wait_scatter(sc, dl, ssem)
        sr_wait(ci, sc, swsem)
        wait_dscatter()
        compute(ci, xs, sc, dl, dd, de, 0)
        start_idx(ci + 2, si, dd, de, isem)
        start_scatter(sc, dl, ssem)
        sr_start(ci, sc, swsem)
        start_dscatter()

    def pair0_body(t, _):
        c0 = 2 * t
        do_chunk0(c0, si0, dd0, de0, ix0, xs0, gx0, sc0, dl0, ss0, sw0,
                  si1, dd1, de1, ix1, xs1, gx1)
        do_chunk0(c0 + 1, si1, dd1, de1, ix1, xs1, gx1, sc1, dl1, ss1, sw1,
                  si0, dd0, de0, ix0, xs0, gx0)
        return 0

    lax.fori_loop(0, (NCHUNK - 1) // 2, pair0_body, 0)
    do_chunk0(NCHUNK - 1, si0, dd0, de0, ix0, xs0, gx0, sc0, dl0, ss0, sw0,
              si1, dd1, de1, ix1, xs1, gx1, last=True)
    wait_idx(NCHUNK - 1, si0, dd0, de0, ix0)
    wait_idx(NCHUNK - 1, si1, dd1, de1, ix1)
    wait_scatter(sc0, dl0, ss0)
    wait_scatter(sc1, dl1, ss1)
    sr_wait(NCHUNK - 1, sc0, sw0)
    sr_wait(NCHUNK - 1, sc1, sw1)
    wait_dscatter()
    plsc.subcore_barrier()

    copy_out_all(0)
    plsc.subcore_barrier()
    zero_rows(dnbuf)
    init_idx(dl0, HALF)
    init_idx(dl1, HALF)
    zero_accum(dnbuf)
    plsc.subcore_barrier()

    def start_read(ci, sc, sem):
        pltpu.async_copy(sr_hbm.at[pl.ds(base + ci * CHUNK, CHUNK)], sc, sem)

    def wait_read(ci, sc, sem):
        pltpu.make_async_copy(
            sr_hbm.at[pl.ds(base + ci * CHUNK, CHUNK)], sc, sem).wait()

    def start_idx1(ci, dd, sem):
        cc = jnp.minimum(ci, NCHUNK - 1)
        pltpu.async_copy(dst_hbm.at[pl.ds(base + cc * CHUNK, CHUNK)], dd, sem)

    def wait_idx1(ci, dd, sem):
        cc = jnp.minimum(ci, NCHUNK - 1)
        pltpu.make_async_copy(dst_hbm.at[pl.ds(base + cc * CHUNK, CHUNK)],
                              dd, sem).wait()

    def compute_dl(ci, dd, dl):
        def group_body(g, _):
            dv = dd[pl.ds(g * 16, 16)] - HALF
            mine = jnp.logical_and(dv >= 0, dv < HALF)
            dl[pl.ds(g * 16, 16)] = jnp.where(mine, dv, HALF)
            return 0

        lax.fori_loop(0, CHUNK // 16, group_body, 0)

    start_scatter(sc0, dl0, ss0)
    start_scatter(sc1, dl1, ss1)
    start_idx1(0, dd0, ix0)
    start_idx1(1, dd1, ix1)
    wait_idx1(0, dd0, ix0)
    wait_scatter(sc0, dl0, ss0)
    start_read(0, sc0, gx0)

    def do_chunk1(ci, dd, isem, sc, dl, ssem, gsem,
                  ndd, nisem, nsc, ndl, nssem, ngsem, last=False):
        wait_read(ci, sc, gsem)
        if not last:
            wait_idx1(ci + 1, ndd, nisem)
            wait_scatter(nsc, ndl, nssem)
            start_read(ci + 1, nsc, ngsem)
        compute_dl(ci, dd, dl)
        start_idx1(ci + 2, dd, isem)
        start_scatter(sc, dl, ssem)

    def pair1_body(t, _):
        c0 = 2 * t
        do_chunk1(c0, dd0, ix0, sc0, dl0, ss0, gx0,
                  dd1, ix1, sc1, dl1, ss1, gx1)
        do_chunk1(c0 + 1, dd1, ix1, sc1, dl1, ss1, gx1,
                  dd0, ix0, sc0, dl0, ss0, gx0)
        return 0

    lax.fori_loop(0, (NCHUNK - 1) // 2, pair1_body, 0)
    do_chunk1(NCHUNK - 1, dd0, ix0, sc0, dl0, ss0, gx0,
              dd1, ix1, sc1, dl1, ss1, gx1, last=True)
    wait_idx1(NCHUNK - 1, dd0, ix0)
    wait_idx1(NCHUNK - 1, dd1, ix1)
    wait_scatter(sc0, dl0, ss0)
    wait_scatter(sc1, dl1, ss1)
    plsc.subcore_barrier()

    copy_out_all(HALF)

    @pl.when(jnp.logical_and(s < DEN_R // 8, c == 0))
    def _():
        pltpu.sync_copy(dacc.at[pl.ds(s * 8, 8)], d0_hbm.at[pl.ds(s * 8, 8)])

    @pl.when(jnp.logical_and(s < DEN_R // 8, c == 1))
    def _():
        pltpu.sync_copy(dacc.at[pl.ds(s * 8, 8)], d1_hbm.at[pl.ds(s * 8, 8)])


def kernel(x, edge_index, params):
    sl = jnp.arange(N, dtype=jnp.int32)
    pad = jnp.zeros((EP - E_TOT,), jnp.int32)
    src = jnp.concatenate([edge_index[0].astype(jnp.int32), sl, pad])
    dst = jnp.concatenate([edge_index[1].astype(jnp.int32), sl, pad])
    xp = jnp.pad(x, ((0, NPAD - N), (0, 0)))

    nl = len(params) // 4
    p0 = p1 = d0 = d1 = bprev = None
    for i in range(nl):
        wl = params[f"Wl{i}"]
        wr = params[f"Wr{i}"]
        att = params[f"att{i}"]
        b = params[f"b{i}"].reshape(1, D)
        if i == 0:
            xl, xr = _mm_first(xp, wl, wr)
        else:
            xl, xr = _combine_mm(p0, p1, d0, d1, bprev, wl, wr)
        e, m = _sc_scores(xl, xr, att, src, dst)
        p0, p1, d0, d1, _ = _sc_aggregate(xl, src, dst, e, m)
        bprev = b
    return _final_pool(p0, p1, d0, d1, bprev)


# speedup vs baseline: 11.4119x; 1.2459x over previous
import functools
import jax
import jax.numpy as jnp
from jax import lax
from jax.experimental import pallas as pl
from jax.experimental.pallas import tpu as pltpu
from jax.experimental.pallas import tpu_sc as plsc

N = 10000
E_RAW = 320000
E_TOT = E_RAW + N
D = 128
NEG = 0.2
NW = 32
CHUNK = 64
NCHUNK = 162
EPT = NCHUNK * CHUNK
EP = NW * EPT
NPAD = 10240
DEN_R = NPAD // D
ROWB = 1024
NROWB = NPAD // ROWB
DEN_RB = DEN_R // NROWB
HALF = NPAD // 2
ACC_R = HALF + 128
SUB_R = ACC_R // 16
OUT_R = HALF // 16

_mesh = plsc.VectorSubcoreMesh(core_axis_name="c", subcore_axis_name="s")


def _bfly_sum(v, lanes):
    for sh in (8, 4, 2, 1):
        v = v + jnp.take_along_axis(v, jnp.bitwise_xor(lanes, sh), axis=0)
    return v


def _bfly_max(v, lanes):
    for sh in (8, 4, 2, 1):
        v = jnp.maximum(
            v, jnp.take_along_axis(v, jnp.bitwise_xor(lanes, sh), axis=0))
    return v


def _den_column(d_blk):
    r = lax.broadcasted_iota(jnp.int32, (ROWB, DEN_RB), 0)
    k = lax.broadcasted_iota(jnp.int32, (ROWB, DEN_RB), 1)
    sel = (k == (r >> 7)).astype(jnp.float32)
    expanded = jnp.dot(sel, d_blk, preferred_element_type=jnp.float32)
    rr = lax.broadcasted_iota(jnp.int32, (ROWB, D), 0)
    cc = lax.broadcasted_iota(jnp.int32, (ROWB, D), 1)
    mask = (cc == (rr & (D - 1))).astype(jnp.float32)
    return jnp.sum(expanded * mask, axis=1, keepdims=True)


def _bounds(xl, xr, att_ref, bl_ref, br_ref):
    aa = jnp.abs(att_ref[...])
    bl = jnp.max(jnp.sum(jnp.abs(xl) * aa, axis=1))
    br = jnp.max(jnp.sum(jnp.abs(xr) * aa, axis=1))
    bl_ref[...] = jnp.full((1, 1, D), bl, jnp.float32)
    br_ref[...] = jnp.full((1, 1, D), br, jnp.float32)


def _mm_first_body(h_ref, wl_ref, wr_ref, att_ref,
                   xl_ref, xr_ref, bl_ref, br_ref):
    h = h_ref[...]
    xl = jnp.dot(h, wl_ref[...], preferred_element_type=jnp.float32)
    xr = jnp.dot(h, wr_ref[...], preferred_element_type=jnp.float32)
    xl_ref[...] = xl
    xr_ref[...] = xr
    _bounds(xl, xr, att_ref, bl_ref, br_ref)


def _mm_first(h, wl, wr, att):
    return pl.pallas_call(
        _mm_first_body,
        grid=(NROWB,),
        in_specs=[
            pl.BlockSpec((ROWB, D), lambda i: (i, 0)),
            pl.BlockSpec((D, D), lambda i: (0, 0)),
            pl.BlockSpec((D, D), lambda i: (0, 0)),
            pl.BlockSpec((1, D), lambda i: (0, 0)),
        ],
        out_specs=[
            pl.BlockSpec((ROWB, D), lambda i: (i, 0)),
            pl.BlockSpec((ROWB, D), lambda i: (i, 0)),
            pl.BlockSpec((1, 1, D), lambda i: (i, 0, 0)),
            pl.BlockSpec((1, 1, D), lambda i: (i, 0, 0)),
        ],
        out_shape=[
            jax.ShapeDtypeStruct((NPAD, D), jnp.float32),
            jax.ShapeDtypeStruct((NPAD, D), jnp.float32),
            jax.ShapeDtypeStruct((NROWB, 1, D), jnp.float32),
            jax.ShapeDtypeStruct((NROWB, 1, D), jnp.float32),
        ],
    )(h, wl, wr, att)


def _combine_body(p0_ref, p1_ref, d0_ref, d1_ref, b_ref):
    den = _den_column(d0_ref[...] + d1_ref[...])
    return (p0_ref[...] + p1_ref[...]) / (den + 1e-16) + b_ref[...]


def _combine_mm_body(p0_ref, p1_ref, d0_ref, d1_ref, b_ref, wl_ref, wr_ref,
                     att_ref, xl_ref, xr_ref, bl_ref, br_ref):
    h = _combine_body(p0_ref, p1_ref, d0_ref, d1_ref, b_ref)
    h = jnp.maximum(h, 0.0)
    xl = jnp.dot(h, wl_ref[...], preferred_element_type=jnp.float32)
    xr = jnp.dot(h, wr_ref[...], preferred_element_type=jnp.float32)
    xl_ref[...] = xl
    xr_ref[...] = xr
    _bounds(xl, xr, att_ref, bl_ref, br_ref)


def _combine_mm(p0, p1, d0, d1, b, wl, wr, att):
    return pl.pallas_call(
        _combine_mm_body,
        grid=(NROWB,),
        in_specs=[
            pl.BlockSpec((ROWB, D), lambda i: (i, 0)),
            pl.BlockSpec((ROWB, D), lambda i: (i, 0)),
            pl.BlockSpec((DEN_RB, D), lambda i: (i, 0)),
            pl.BlockSpec((DEN_RB, D), lambda i: (i, 0)),
            pl.BlockSpec((1, D), lambda i: (0, 0)),
            pl.BlockSpec((D, D), lambda i: (0, 0)),
            pl.BlockSpec((D, D), lambda i: (0, 0)),
            pl.BlockSpec((1, D), lambda i: (0, 0)),
        ],
        out_specs=[
            pl.BlockSpec((ROWB, D), lambda i: (i, 0)),
            pl.BlockSpec((ROWB, D), lambda i: (i, 0)),
            pl.BlockSpec((1, 1, D), lambda i: (i, 0, 0)),
            pl.BlockSpec((1, 1, D), lambda i: (i, 0, 0)),
        ],
        out_shape=[
            jax.ShapeDtypeStruct((NPAD, D), jnp.float32),
            jax.ShapeDtypeStruct((NPAD, D), jnp.float32),
            jax.ShapeDtypeStruct((NROWB, 1, D), jnp.float32),
            jax.ShapeDtypeStruct((NROWB, 1, D), jnp.float32),
        ],
    )(p0, p1, d0, d1, b, wl, wr, att)


def _final_body(p0_ref, p1_ref, d0_ref, d1_ref, b_ref, o_ref):
    i = pl.program_id(0)
    h = _combine_body(p0_ref, p1_ref, d0_ref, d1_ref, b_ref)
    gi = i * ROWB + lax.broadcasted_iota(jnp.int32, (ROWB, D), 0)
    h = jnp.where(gi < N, h, 0.0)
    s = jnp.sum(h, axis=0, keepdims=True) * (1.0 / N)

    @pl.when(i == 0)
    def _():
        o_ref[...] = s

    @pl.when(i > 0)
    def _():
        o_ref[...] += s


def _final_pool(p0, p1, d0, d1, b):
    return pl.pallas_call(
        _final_body,
        grid=(NROWB,),
        in_specs=[
            pl.BlockSpec((ROWB, D), lambda i: (i, 0)),
            pl.BlockSpec((ROWB, D), lambda i: (i, 0)),
            pl.BlockSpec((DEN_RB, D), lambda i: (i, 0)),
            pl.BlockSpec((DEN_RB, D), lambda i: (i, 0)),
            pl.BlockSpec((1, D), lambda i: (0, 0)),
        ],
        out_specs=pl.BlockSpec((1, D), lambda i: (0, 0)),
        out_shape=jax.ShapeDtypeStruct((1, D), jnp.float32),
    )(p0, p1, d0, d1, b)


@functools.partial(
    pl.kernel,
    out_type=[
        jax.ShapeDtypeStruct((NPAD, D), jnp.float32),
        jax.ShapeDtypeStruct((NPAD, D), jnp.float32),
        jax.ShapeDtypeStruct((DEN_R, D), jnp.float32),
        jax.ShapeDtypeStruct((DEN_R, D), jnp.float32),
        jax.ShapeDtypeStruct((EP + CHUNK, D), jnp.float32),
    ],
    mesh=_mesh,
    scratch_types=[
        pltpu.VMEM((CHUNK,), jnp.int32),
        pltpu.VMEM((CHUNK,), jnp.int32),
        pltpu.VMEM((CHUNK,), jnp.int32),
        pltpu.VMEM((CHUNK,), jnp.int32),
        pltpu.VMEM((CHUNK,), jnp.int32),
        pltpu.VMEM((CHUNK,), jnp.int32),
        pltpu.VMEM((CHUNK,), jnp.int32),
        pltpu.VMEM((CHUNK, D), jnp.float32),
        pltpu.VMEM((CHUNK, D), jnp.float32),
        pltpu.VMEM((CHUNK, D), jnp.float32),
        pltpu.VMEM((CHUNK, D), jnp.float32),
        pltpu.VMEM((CHUNK, D), jnp.float32),
        pltpu.VMEM((CHUNK, D), jnp.float32),
        pltpu.VMEM((CHUNK, D), jnp.float32),
        pltpu.VMEM((D,), jnp.float32),
        pltpu.VMEM((NROWB, 1, D), jnp.float32),
        pltpu.VMEM((NROWB, 1, D), jnp.float32),
        pltpu.VMEM_SHARED((ACC_R, D), jnp.float32),
        pltpu.VMEM_SHARED((DEN_R, D), jnp.float32),
        pltpu.SemaphoreType.DMA,
        pltpu.SemaphoreType.DMA,
        pltpu.SemaphoreType.DMA,
        pltpu.SemaphoreType.DMA,
        pltpu.SemaphoreType.DMA,
        pltpu.SemaphoreType.DMA,
        pltpu.SemaphoreType.DMA,
        pltpu.SemaphoreType.DMA,
        pltpu.SemaphoreType.DMA,
        pltpu.SemaphoreType.DMA,
        pltpu.SemaphoreType.DMA,
    ],
)
def _sc_layer(xl_hbm, xr_hbm, att_hbm, bl_hbm, br_hbm, src_hbm, dst_hbm,
              p0_hbm, p1_hbm, d0_hbm, d1_hbm, sr_hbm,
              si0, si1, dd0, dd1, dl0, dl1, bidx,
              xs0, xs1, xq0, xq1, sc0, sc1, dnbuf, attv, blv, brv,
              accum, dacc,
              gx0, gx1, gr0, gr1, ss0, ss1, sd, ix0, ix1, sw0, sw1):
    c = lax.axis_index("c")
    s = lax.axis_index("s")
    wid = s * 2 + c
    base = wid * EPT
    lanes = lax.iota(jnp.int32, 16)

    pltpu.sync_copy(att_hbm, attv)
    pltpu.sync_copy(bl_hbm, blv)
    pltpu.sync_copy(br_hbm, brv)
    cl = blv[0, 0, pl.ds(0, 16)]
    cr = brv[0, 0, pl.ds(0, 16)]
    for i in range(1, NROWB):
        cl = jnp.maximum(cl, blv[i, 0, pl.ds(0, 16)])
        cr = jnp.maximum(cr, brv[i, 0, pl.ds(0, 16)])
    c16 = cl + cr

    def init_idx(buf, val):
        def ib(g, _):
            buf[pl.ds(g * 16, 16)] = jnp.full((16,), val, jnp.int32)
            return 0
        lax.fori_loop(0, CHUNK // 16, ib, 0)

    def zero_rows(buf):
        def zr(e, _):
            for j in range(D // 16):
                buf[e, pl.ds(j * 16, 16)] = jnp.zeros((16,), jnp.float32)
            return 0
        lax.fori_loop(0, CHUNK, zr, 0)

    def zero_accum(zsrc):
        for k in range(SUB_R // CHUNK):
            pltpu.sync_copy(zsrc,
                            accum.at[pl.ds(s * SUB_R + k * CHUNK, CHUNK)])
        rem = SUB_R - (SUB_R // CHUNK) * CHUNK
        if rem:
            pltpu.sync_copy(
                zsrc.at[pl.ds(0, rem)],
                accum.at[pl.ds(s * SUB_R + SUB_R - rem, rem)])

    def start_idx(ci, si, dd, sem):
        cc = jnp.minimum(ci, NCHUNK - 1)
        off = base + cc * CHUNK
        pltpu.async_copy(src_hbm.at[pl.ds(off, CHUNK)], si, sem)
        pltpu.async_copy(dst_hbm.at[pl.ds(off, CHUNK)], dd, sem)

    def wait_idx(ci, si, dd, sem):
        cc = jnp.minimum(ci, NCHUNK - 1)
        off = base + cc * CHUNK
        pltpu.make_async_copy(src_hbm.at[pl.ds(off, CHUNK)], si, sem).wait()
        pltpu.make_async_copy(dst_hbm.at[pl.ds(off, CHUNK)], dd, sem).wait()

    def start_gather(si, dd, xs, xq, semxs, semxq):
        pltpu.async_copy(xl_hbm.at[si], xs, semxs)
        pltpu.async_copy(xr_hbm.at[dd], xq, semxq)

    def wait_gather(si, dd, xs, xq, semxs, semxq):
        pltpu.make_async_copy(xl_hbm.at[si], xs, semxs).wait()
        pltpu.make_async_copy(xr_hbm.at[dd], xq, semxq).wait()

    def start_scatter(sc, dl, sem):
        pltpu.async_copy(sc, accum.at[dl], sem, add=True)

    def wait_scatter(sc, dl, sem):
        pltpu.make_async_copy(sc, accum.at[dl], sem).wait()

    def start_dscatter():
        pltpu.async_copy(dnbuf, dacc.at[bidx], sd, add=True)

    def wait_dscatter():
        pltpu.make_async_copy(dnbuf, dacc.at[bidx], sd).wait()

    def sr_start(ci, sc, sem):
        pltpu.async_copy(sc, sr_hbm.at[pl.ds(base + ci * CHUNK, CHUNK)], sem)

    def sr_wait(ci, sc, sem):
        pltpu.make_async_copy(
            sc, sr_hbm.at[pl.ds(base + ci * CHUNK, CHUNK)], sem).wait()

    def sr_prime(sc, sem):
        pltpu.async_copy(sc, sr_hbm.at[pl.ds(EP, CHUNK)], sem)

    def copy_out(nlo, rlo, nrows):
        rows = accum.at[pl.ds(rlo, nrows)]

        @pl.when(c == 0)
        def _():
            pltpu.sync_copy(rows, p0_hbm.at[pl.ds(nlo + rlo, nrows)])

        @pl.when(c == 1)
        def _():
            pltpu.sync_copy(rows, p1_hbm.at[pl.ds(nlo + rlo, nrows)])

    def copy_out_all(nlo):
        for k in range(OUT_R // CHUNK):
            copy_out(nlo, s * OUT_R + k * CHUNK, CHUNK)

    zero_rows(sc0)
    zero_rows(sc1)
    zero_rows(dnbuf)
    init_idx(dl0, HALF)
    init_idx(dl1, HALF)
    init_idx(bidx, 0)
    zero_accum(sc0)

    @pl.when(s == 0)
    def _():
        pltpu.sync_copy(sc0.at[pl.ds(0, CHUNK)], dacc.at[pl.ds(0, CHUNK)])
        pltpu.sync_copy(sc0.at[pl.ds(0, DEN_R - CHUNK)],
                        dacc.at[pl.ds(CHUNK, DEN_R - CHUNK)])

    plsc.subcore_barrier()

    def compute0(ci, xs, xq, sc, dl, dd):
        lo = ci * CHUNK

        def group_body(g, _):
            @plsc.parallel_loop(0, 16, unroll=8,
                                carry=jnp.zeros((16,), jnp.float32))
            def e16(k, e16i):
                e = g * 16 + k
                acc = jnp.zeros((16,), jnp.float32)
                for j in range(8):
                    a = xs[e, pl.ds(j * 16, 16)] + xq[e, pl.ds(j * 16, 16)]
                    lk = jnp.maximum(a, NEG * a)
                    acc = acc + attv[pl.ds(j * 16, 16)] * lk
                return jnp.where(lanes == k, _bfly_sum(acc, lanes), e16i)

            gi = base + lo + g * 16 + lanes
            p16 = jnp.where(gi < E_TOT, jnp.exp(e16 - c16), 0.0)
            dvg = dd[pl.ds(g * 16, 16)]
            mine = dvg < HALF
            dl[pl.ds(g * 16, 16)] = jnp.where(mine, dvg, HALF)
            bidx[pl.ds(g * 16, 16)] = dvg >> 7
            dm = dvg & (D - 1)

            @plsc.parallel_loop(0, 16, unroll=8)
            def _(k):
                e = g * 16 + k
                kk = jnp.full((16,), k, jnp.int32)
                pvec = jnp.take_along_axis(p16, kk, axis=0)
                dmk = jnp.take_along_axis(dm, kk, axis=0)
                for j in range(8):
                    sc[e, pl.ds(j * 16, 16)] = \
                        pvec * xs[e, pl.ds(j * 16, 16)]
                    dnbuf[e, pl.ds(j * 16, 16)] = \
                        jnp.where(lanes + (j * 16) == dmk, pvec, 0.0)

            return 0

        lax.fori_loop(0, CHUNK // 16, group_body, 0)

    start_scatter(sc0, dl0, ss0)
    start_scatter(sc1, dl1, ss1)
    start_dscatter()
    sr_prime(sc0, sw0)
    sr_prime(sc1, sw1)
    start_idx(0, si0, dd0, ix0)
    start_idx(1, si1, dd1, ix1)
    wait_idx(0, si0, dd0, ix0)
    start_gather(si0, dd0, xs0, xq0, gx0, gr0)

    def do_chunk0(ci, si, dd, isem, xs, xq, gsem, qsem, sc, dl, ssem, swsem,
                  nsi, ndd, nisem, nxs, nxq, ngsem, nqsem, last=False):
        wait_gather(si, dd, xs, xq, gsem, qsem)
        if not last:
            wait_idx(ci + 1, nsi, ndd, nisem)
            start_gather(nsi, ndd, nxs, nxq, ngsem, nqsem)
        wait_scatter(sc, dl, ssem)
        sr_wait(ci, sc, swsem)
        wait_dscatter()
        compute0(ci, xs, xq, sc, dl, dd)
        start_idx(ci + 2, si, dd, isem)
        start_scatter(sc, dl, ssem)
        sr_start(ci, sc, swsem)
        start_dscatter()

    def pair0_body(t, _):
        c0 = 2 * t
        do_chunk0(c0, si0, dd0, ix0, xs0, xq0, gx0, gr0, sc0, dl0, ss0, sw0,
                  si1, dd1, ix1, xs1, xq1, gx1, gr1)
        do_chunk0(c0 + 1, si1, dd1, ix1, xs1, xq1, gx1, gr1,
                  sc1, dl1, ss1, sw1,
                  si0, dd0, ix0, xs0, xq0, gx0, gr0)
        return 0

    lax.fori_loop(0, NCHUNK // 2 - 1, pair0_body, 0)
    do_chunk0(NCHUNK - 2, si0, dd0, ix0, xs0, xq0, gx0, gr0,
              sc0, dl0, ss0, sw0,
              si1, dd1, ix1, xs1, xq1, gx1, gr1)
    do_chunk0(NCHUNK - 1, si1, dd1, ix1, xs1, xq1, gx1, gr1,
              sc1, dl1, ss1, sw1,
              si0, dd0, ix0, xs0, xq0, gx0, gr0, last=True)
    wait_idx(NCHUNK - 1, si0, dd0, ix0)
    wait_idx(NCHUNK - 1, si1, dd1, ix1)
    wait_scatter(sc0, dl0, ss0)
    wait_scatter(sc1, dl1, ss1)
    sr_wait(NCHUNK - 1, sc0, sw0)
    sr_wait(NCHUNK - 1, sc1, sw1)
    wait_dscatter()
    plsc.subcore_barrier()

    copy_out_all(0)
    plsc.subcore_barrier()
    zero_rows(dnbuf)
    init_idx(dl0, HALF)
    init_idx(dl1, HALF)
    zero_accum(dnbuf)
    plsc.subcore_barrier()

    def start_read(ci, sc, sem):
        pltpu.async_copy(sr_hbm.at[pl.ds(base + ci * CHUNK, CHUNK)], sc, sem)

    def wait_read(ci, sc, sem):
        pltpu.make_async_copy(
            sr_hbm.at[pl.ds(base + ci * CHUNK, CHUNK)], sc, sem).wait()

    def start_idx1(ci, dd, sem):
        cc = jnp.minimum(ci, NCHUNK - 1)
        pltpu.async_copy(dst_hbm.at[pl.ds(base + cc * CHUNK, CHUNK)], dd, sem)

    def wait_idx1(ci, dd, sem):
        cc = jnp.minimum(ci, NCHUNK - 1)
        pltpu.make_async_copy(dst_hbm.at[pl.ds(base + cc * CHUNK, CHUNK)],
                              dd, sem).wait()

    def compute_dl(ci, dd, dl):
        def group_body(g, _):
            dv = dd[pl.ds(g * 16, 16)] - HALF
            mine = dv >= 0
            dl[pl.ds(g * 16, 16)] = jnp.where(mine, dv, HALF)
            return 0

        lax.fori_loop(0, CHUNK // 16, group_body, 0)

    start_scatter(sc0, dl0, ss0)
    start_scatter(sc1, dl1, ss1)
    start_idx1(0, dd0, ix0)
    start_idx1(1, dd1, ix1)
    wait_idx1(0, dd0, ix0)
    wait_scatter(sc0, dl0, ss0)
    start_read(0, sc0, gx0)

    def do_chunk1(ci, dd, isem, sc, dl, ssem, gsem,
                  ndd, nisem, nsc, ndl, nssem, ngsem, last=False):
        wait_read(ci, sc, gsem)
        if not last:
            wait_idx1(ci + 1, ndd, nisem)
            wait_scatter(nsc, ndl, nssem)
            start_read(ci + 1, nsc, ngsem)
        compute_dl(ci, dd, dl)
        start_idx1(ci + 2, dd, isem)
        start_scatter(sc, dl, ssem)

    def pair1_body(t, _):
        c0 = 2 * t
        do_chunk1(c0, dd0, ix0, sc0, dl0, ss0, gx0,
                  dd1, ix1, sc1, dl1, ss1, gx1)
        do_chunk1(c0 + 1, dd1, ix1, sc1, dl1, ss1, gx1,
                  dd0, ix0, sc0, dl0, ss0, gx0)
        return 0

    lax.fori_loop(0, NCHUNK // 2 - 1, pair1_body, 0)
    do_chunk1(NCHUNK - 2, dd0, ix0, sc0, dl0, ss0, gx0,
              dd1, ix1, sc1, dl1, ss1, gx1)
    do_chunk1(NCHUNK - 1, dd1, ix1, sc1, dl1, ss1, gx1,
              dd0, ix0, sc0, dl0, ss0, gx0, last=True)
    wait_idx1(NCHUNK - 1, dd0, ix0)
    wait_idx1(NCHUNK - 1, dd1, ix1)
    wait_scatter(sc0, dl0, ss0)
    wait_scatter(sc1, dl1, ss1)
    plsc.subcore_barrier()

    copy_out_all(HALF)

    @pl.when(jnp.logical_and(s < DEN_R // 8, c == 0))
    def _():
        pltpu.sync_copy(dacc.at[pl.ds(s * 8, 8)], d0_hbm.at[pl.ds(s * 8, 8)])

    @pl.when(jnp.logical_and(s < DEN_R // 8, c == 1))
    def _():
        pltpu.sync_copy(dacc.at[pl.ds(s * 8, 8)], d1_hbm.at[pl.ds(s * 8, 8)])


def kernel(x, edge_index, params):
    sl = jnp.arange(N, dtype=jnp.int32)
    pad = jnp.zeros((EP - E_TOT,), jnp.int32)
    src = jnp.concatenate([edge_index[0].astype(jnp.int32), sl, pad])
    dst = jnp.concatenate([edge_index[1].astype(jnp.int32), sl, pad])
    xp = jnp.pad(x, ((0, NPAD - N), (0, 0)))

    nl = len(params) // 4
    p0 = p1 = d0 = d1 = bprev = None
    for i in range(nl):
        wl = params[f"Wl{i}"]
        wr = params[f"Wr{i}"]
        att = params[f"att{i}"].reshape(1, D)
        b = params[f"b{i}"].reshape(1, D)
        if i == 0:
            xl, xr, bl, br = _mm_first(xp, wl, wr, att)
        else:
            xl, xr, bl, br = _combine_mm(p0, p1, d0, d1, bprev, wl, wr, att)
        p0, p1, d0, d1, _ = _sc_layer(xl, xr, params[f"att{i}"], bl, br,
                                      src, dst)
        bprev = b
    return _final_pool(p0, p1, d0, d1, bprev)


# speedup vs baseline: 11.4262x; 1.0013x over previous
import functools
import jax
import jax.numpy as jnp
from jax import lax
from jax.experimental import pallas as pl
from jax.experimental.pallas import tpu as pltpu
from jax.experimental.pallas import tpu_sc as plsc

N = 10000
E_RAW = 320000
E_TOT = E_RAW + N
D = 128
NEG = 0.2
NW = 32
CHUNK = 64
NCHUNK = 162
EPT = NCHUNK * CHUNK
EP = NW * EPT
NPAD = 10240
DEN_R = NPAD // D
ROWB = 1024
NROWB = NPAD // ROWB
DEN_RB = DEN_R // NROWB
HALF = NPAD // 2
ACC_R = HALF + 128
SUB_R = ACC_R // 16
OUT_R = HALF // 16

_mesh = plsc.VectorSubcoreMesh(core_axis_name="c", subcore_axis_name="s")


def _bfly_sum(v, lanes):
    for sh in (8, 4, 2, 1):
        v = v + jnp.take_along_axis(v, jnp.bitwise_xor(lanes, sh), axis=0)
    return v


def _bfly_max(v, lanes):
    for sh in (8, 4, 2, 1):
        v = jnp.maximum(
            v, jnp.take_along_axis(v, jnp.bitwise_xor(lanes, sh), axis=0))
    return v


def _den_column(d_blk):
    r = lax.broadcasted_iota(jnp.int32, (ROWB, DEN_RB), 0)
    k = lax.broadcasted_iota(jnp.int32, (ROWB, DEN_RB), 1)
    sel = (k == (r >> 7)).astype(jnp.float32)
    expanded = jnp.dot(sel, d_blk, preferred_element_type=jnp.float32)
    rr = lax.broadcasted_iota(jnp.int32, (ROWB, D), 0)
    cc = lax.broadcasted_iota(jnp.int32, (ROWB, D), 1)
    mask = (cc == (rr & (D - 1))).astype(jnp.float32)
    return jnp.sum(expanded * mask, axis=1, keepdims=True)


def _bounds(xl, xr, att_ref, bl_ref, br_ref):
    aa = jnp.abs(att_ref[...])
    bl = jnp.max(jnp.sum(jnp.abs(xl) * aa, axis=1))
    br = jnp.max(jnp.sum(jnp.abs(xr) * aa, axis=1))
    bl_ref[...] = jnp.full((1, 1, D), bl, jnp.float32)
    br_ref[...] = jnp.full((1, 1, D), br, jnp.float32)


def _mm_first_body(h_ref, wl_ref, wr_ref, att_ref,
                   xl_ref, xr_ref, bl_ref, br_ref):
    h = h_ref[...]
    xl = jnp.dot(h, wl_ref[...], preferred_element_type=jnp.float32)
    xr = jnp.dot(h, wr_ref[...], preferred_element_type=jnp.float32)
    xl_ref[...] = xl
    xr_ref[...] = xr
    _bounds(xl, xr, att_ref, bl_ref, br_ref)


def _mm_first(h, wl, wr, att):
    return pl.pallas_call(
        _mm_first_body,
        grid=(NROWB,),
        in_specs=[
            pl.BlockSpec((ROWB, D), lambda i: (i, 0)),
            pl.BlockSpec((D, D), lambda i: (0, 0)),
            pl.BlockSpec((D, D), lambda i: (0, 0)),
            pl.BlockSpec((1, D), lambda i: (0, 0)),
        ],
        out_specs=[
            pl.BlockSpec((ROWB, D), lambda i: (i, 0)),
            pl.BlockSpec((ROWB, D), lambda i: (i, 0)),
            pl.BlockSpec((1, 1, D), lambda i: (i, 0, 0)),
            pl.BlockSpec((1, 1, D), lambda i: (i, 0, 0)),
        ],
        out_shape=[
            jax.ShapeDtypeStruct((NPAD, D), jnp.float32),
            jax.ShapeDtypeStruct((NPAD, D), jnp.float32),
            jax.ShapeDtypeStruct((NROWB, 1, D), jnp.float32),
            jax.ShapeDtypeStruct((NROWB, 1, D), jnp.float32),
        ],
    )(h, wl, wr, att)


def _combine_body(p0_ref, p1_ref, d0_ref, d1_ref, b_ref):
    den = _den_column(d0_ref[...] + d1_ref[...])
    return (p0_ref[...] + p1_ref[...]) / (den + 1e-16) + b_ref[...]


def _combine_mm_body(p0_ref, p1_ref, d0_ref, d1_ref, b_ref, wl_ref, wr_ref,
                     att_ref, xl_ref, xr_ref, bl_ref, br_ref):
    h = _combine_body(p0_ref, p1_ref, d0_ref, d1_ref, b_ref)
    h = jnp.maximum(h, 0.0)
    xl = jnp.dot(h, wl_ref[...], preferred_element_type=jnp.float32)
    xr = jnp.dot(h, wr_ref[...], preferred_element_type=jnp.float32)
    xl_ref[...] = xl
    xr_ref[...] = xr
    _bounds(xl, xr, att_ref, bl_ref, br_ref)


def _combine_mm(p0, p1, d0, d1, b, wl, wr, att):
    return pl.pallas_call(
        _combine_mm_body,
        grid=(NROWB,),
        in_specs=[
            pl.BlockSpec((ROWB, D), lambda i: (i, 0)),
            pl.BlockSpec((ROWB, D), lambda i: (i, 0)),
            pl.BlockSpec((DEN_RB, D), lambda i: (i, 0)),
            pl.BlockSpec((DEN_RB, D), lambda i: (i, 0)),
            pl.BlockSpec((1, D), lambda i: (0, 0)),
            pl.BlockSpec((D, D), lambda i: (0, 0)),
            pl.BlockSpec((D, D), lambda i: (0, 0)),
            pl.BlockSpec((1, D), lambda i: (0, 0)),
        ],
        out_specs=[
            pl.BlockSpec((ROWB, D), lambda i: (i, 0)),
            pl.BlockSpec((ROWB, D), lambda i: (i, 0)),
            pl.BlockSpec((1, 1, D), lambda i: (i, 0, 0)),
            pl.BlockSpec((1, 1, D), lambda i: (i, 0, 0)),
        ],
        out_shape=[
            jax.ShapeDtypeStruct((NPAD, D), jnp.float32),
            jax.ShapeDtypeStruct((NPAD, D), jnp.float32),
            jax.ShapeDtypeStruct((NROWB, 1, D), jnp.float32),
            jax.ShapeDtypeStruct((NROWB, 1, D), jnp.float32),
        ],
    )(p0, p1, d0, d1, b, wl, wr, att)


def _final_body(p0_ref, p1_ref, d0_ref, d1_ref, b_ref, o_ref):
    i = pl.program_id(0)
    h = _combine_body(p0_ref, p1_ref, d0_ref, d1_ref, b_ref)
    gi = i * ROWB + lax.broadcasted_iota(jnp.int32, (ROWB, D), 0)
    h = jnp.where(gi < N, h, 0.0)
    s = jnp.sum(h, axis=0, keepdims=True) * (1.0 / N)

    @pl.when(i == 0)
    def _():
        o_ref[...] = s

    @pl.when(i > 0)
    def _():
        o_ref[...] += s


def _final_pool(p0, p1, d0, d1, b):
    return pl.pallas_call(
        _final_body,
        grid=(NROWB,),
        in_specs=[
            pl.BlockSpec((ROWB, D), lambda i: (i, 0)),
            pl.BlockSpec((ROWB, D), lambda i: (i, 0)),
            pl.BlockSpec((DEN_RB, D), lambda i: (i, 0)),
            pl.BlockSpec((DEN_RB, D), lambda i: (i, 0)),
            pl.BlockSpec((1, D), lambda i: (0, 0)),
        ],
        out_specs=pl.BlockSpec((1, D), lambda i: (0, 0)),
        out_shape=jax.ShapeDtypeStruct((1, D), jnp.float32),
    )(p0, p1, d0, d1, b)


@functools.partial(
    pl.kernel,
    out_type=[
        jax.ShapeDtypeStruct((NPAD, D), jnp.float32),
        jax.ShapeDtypeStruct((NPAD, D), jnp.float32),
        jax.ShapeDtypeStruct((DEN_R, D), jnp.float32),
        jax.ShapeDtypeStruct((DEN_R, D), jnp.float32),
        jax.ShapeDtypeStruct((EP + CHUNK, D), jnp.float32),
    ],
    mesh=_mesh,
    scratch_types=[
        pltpu.VMEM((CHUNK,), jnp.int32),
        pltpu.VMEM((CHUNK,), jnp.int32),
        pltpu.VMEM((CHUNK,), jnp.int32),
        pltpu.VMEM((CHUNK,), jnp.int32),
        pltpu.VMEM((CHUNK,), jnp.int32),
        pltpu.VMEM((CHUNK,), jnp.int32),
        pltpu.VMEM((CHUNK,), jnp.int32),
        pltpu.VMEM((CHUNK, D), jnp.float32),
        pltpu.VMEM((CHUNK, D), jnp.float32),
        pltpu.VMEM((CHUNK, D), jnp.float32),
        pltpu.VMEM((CHUNK, D), jnp.float32),
        pltpu.VMEM((CHUNK, D), jnp.float32),
        pltpu.VMEM((CHUNK, D), jnp.float32),
        pltpu.VMEM((CHUNK, D), jnp.float32),
        pltpu.VMEM((D,), jnp.float32),
        pltpu.VMEM((NROWB, 1, D), jnp.float32),
        pltpu.VMEM((NROWB, 1, D), jnp.float32),
        pltpu.VMEM_SHARED((ACC_R, D), jnp.float32),
        pltpu.VMEM_SHARED((DEN_R, D), jnp.float32),
        pltpu.SemaphoreType.DMA,
        pltpu.SemaphoreType.DMA,
        pltpu.SemaphoreType.DMA,
        pltpu.SemaphoreType.DMA,
        pltpu.SemaphoreType.DMA,
        pltpu.SemaphoreType.DMA,
        pltpu.SemaphoreType.DMA,
        pltpu.SemaphoreType.DMA,
        pltpu.SemaphoreType.DMA,
        pltpu.SemaphoreType.DMA,
        pltpu.SemaphoreType.DMA,
    ],
)
def _sc_layer(xl_hbm, xr_hbm, att_hbm, bl_hbm, br_hbm, src_hbm, dst_hbm,
              p0_hbm, p1_hbm, d0_hbm, d1_hbm, sr_hbm,
              si0, si1, dd0, dd1, dl0, dl1, bidx,
              xs0, xs1, xq0, xq1, sc0, sc1, dnbuf, attv, blv, brv,
              accum, dacc,
              gx0, gx1, gr0, gr1, ss0, ss1, sd, ix0, ix1, sw0, sw1):
    c = lax.axis_index("c")
    s = lax.axis_index("s")
    wid = s * 2 + c
    base = wid * EPT
    lanes = lax.iota(jnp.int32, 16)

    pltpu.sync_copy(att_hbm, attv)
    pltpu.sync_copy(bl_hbm, blv)
    pltpu.sync_copy(br_hbm, brv)
    cl = blv[0, 0, pl.ds(0, 16)]
    cr = brv[0, 0, pl.ds(0, 16)]
    for i in range(1, NROWB):
        cl = jnp.maximum(cl, blv[i, 0, pl.ds(0, 16)])
        cr = jnp.maximum(cr, brv[i, 0, pl.ds(0, 16)])
    c16 = cl + cr

    def init_idx(buf, val):
        def ib(g, _):
            buf[pl.ds(g * 16, 16)] = jnp.full((16,), val, jnp.int32)
            return 0
        lax.fori_loop(0, CHUNK // 16, ib, 0)

    def zero_rows(buf):
        def zr(e, _):
            for j in range(D // 16):
                buf[e, pl.ds(j * 16, 16)] = jnp.zeros((16,), jnp.float32)
            return 0
        lax.fori_loop(0, CHUNK, zr, 0)

    def zero_accum(zsrc):
        for k in range(SUB_R // CHUNK):
            pltpu.sync_copy(zsrc,
                            accum.at[pl.ds(s * SUB_R + k * CHUNK, CHUNK)])
        rem = SUB_R - (SUB_R // CHUNK) * CHUNK
        if rem:
            pltpu.sync_copy(
                zsrc.at[pl.ds(0, rem)],
                accum.at[pl.ds(s * SUB_R + SUB_R - rem, rem)])

    def start_idx(ci, si, dd, sem):
        cc = jnp.minimum(ci, NCHUNK - 1)
        off = base + cc * CHUNK
        pltpu.async_copy(src_hbm.at[pl.ds(off, CHUNK)], si, sem)
        pltpu.async_copy(dst_hbm.at[pl.ds(off, CHUNK)], dd, sem)

    def wait_idx(ci, si, dd, sem):
        cc = jnp.minimum(ci, NCHUNK - 1)
        off = base + cc * CHUNK
        pltpu.make_async_copy(src_hbm.at[pl.ds(off, CHUNK)], si, sem).wait()
        pltpu.make_async_copy(dst_hbm.at[pl.ds(off, CHUNK)], dd, sem).wait()

    def start_gather(si, dd, xs, xq, semxs, semxq):
        pltpu.async_copy(xl_hbm.at[si], xs, semxs)
        pltpu.async_copy(xr_hbm.at[dd], xq, semxq)

    def wait_gather(si, dd, xs, xq, semxs, semxq):
        pltpu.make_async_copy(xl_hbm.at[si], xs, semxs).wait()
        pltpu.make_async_copy(xr_hbm.at[dd], xq, semxq).wait()

    def start_scatter(sc, dl, sem):
        pltpu.async_copy(sc, accum.at[dl], sem, add=True)

    def wait_scatter(sc, dl, sem):
        pltpu.make_async_copy(sc, accum.at[dl], sem).wait()

    def start_dscatter():
        pltpu.async_copy(dnbuf, dacc.at[bidx], sd, add=True)

    def wait_dscatter():
        pltpu.make_async_copy(dnbuf, dacc.at[bidx], sd).wait()

    def sr_start(ci, sc, sem):
        pltpu.async_copy(sc, sr_hbm.at[pl.ds(base + ci * CHUNK, CHUNK)], sem)

    def sr_wait(ci, sc, sem):
        pltpu.make_async_copy(
            sc, sr_hbm.at[pl.ds(base + ci * CHUNK, CHUNK)], sem).wait()

    def sr_prime(sc, sem):
        pltpu.async_copy(sc, sr_hbm.at[pl.ds(EP, CHUNK)], sem)

    def copy_out(nlo, rlo, nrows):
        rows = accum.at[pl.ds(rlo, nrows)]

        @pl.when(c == 0)
        def _():
            pltpu.sync_copy(rows, p0_hbm.at[pl.ds(nlo + rlo, nrows)])

        @pl.when(c == 1)
        def _():
            pltpu.sync_copy(rows, p1_hbm.at[pl.ds(nlo + rlo, nrows)])

    def copy_out_all(nlo):
        for k in range(OUT_R // CHUNK):
            copy_out(nlo, s * OUT_R + k * CHUNK, CHUNK)

    zero_rows(sc0)
    zero_rows(sc1)
    zero_rows(dnbuf)
    init_idx(dl0, HALF)
    init_idx(dl1, HALF)
    init_idx(bidx, 0)
    zero_accum(sc0)

    @pl.when(s == 0)
    def _():
        pltpu.sync_copy(sc0.at[pl.ds(0, CHUNK)], dacc.at[pl.ds(0, CHUNK)])
        pltpu.sync_copy(sc0.at[pl.ds(0, DEN_R - CHUNK)],
                        dacc.at[pl.ds(CHUNK, DEN_R - CHUNK)])

    plsc.subcore_barrier()

    def compute0(ci, xs, xq, sc, dl, dd):
        lo = ci * CHUNK

        def group_body(g, _):
            dvg = dd[pl.ds(g * 16, 16)]
            dl[pl.ds(g * 16, 16)] = jnp.where(dvg < HALF, dvg, HALF)
            bidx[pl.ds(g * 16, 16)] = dvg >> 7
            dm = dvg & (D - 1)
            gbase = base + lo + g * 16

            @plsc.parallel_loop(0, 16, unroll=8)
            def _(k):
                e = g * 16 + k
                acc = jnp.zeros((16,), jnp.float32)
                for j in range(8):
                    a = xs[e, pl.ds(j * 16, 16)] + xq[e, pl.ds(j * 16, 16)]
                    lk = jnp.maximum(a, NEG * a)
                    acc = acc + attv[pl.ds(j * 16, 16)] * lk
                pv = jnp.exp(_bfly_sum(acc, lanes) - c16)
                pv = jnp.where(gbase + k < E_TOT, pv,
                               jnp.zeros((16,), jnp.float32))
                dmk = jnp.take_along_axis(
                    dm, jnp.full((16,), k, jnp.int32), axis=0)
                for j in range(8):
                    sc[e, pl.ds(j * 16, 16)] = \
                        pv * xs[e, pl.ds(j * 16, 16)]
                    dnbuf[e, pl.ds(j * 16, 16)] = \
                        jnp.where(lanes + (j * 16) == dmk, pv, 0.0)

            return 0

        lax.fori_loop(0, CHUNK // 16, group_body, 0)

    start_scatter(sc0, dl0, ss0)
    start_scatter(sc1, dl1, ss1)
    start_dscatter()
    sr_prime(sc0, sw0)
    sr_prime(sc1, sw1)
    start_idx(0, si0, dd0, ix0)
    start_idx(1, si1, dd1, ix1)
    wait_idx(0, si0, dd0, ix0)
    start_gather(si0, dd0, xs0, xq0, gx0, gr0)

    def do_chunk0(ci, si, dd, isem, xs, xq, gsem, qsem, sc, dl, ssem, swsem,
                  nsi, ndd, nisem, nxs, nxq, ngsem, nqsem, last=False):
        wait_gather(si, dd, xs, xq, gsem, qsem)
        if not last:
            wait_idx(ci + 1, nsi, ndd, nisem)
            start_gather(nsi, ndd, nxs, nxq, ngsem, nqsem)
        wait_scatter(sc, dl, ssem)
        sr_wait(ci, sc, swsem)
        wait_dscatter()
        compute0(ci, xs, xq, sc, dl, dd)
        start_idx(ci + 2, si, dd, isem)
        start_scatter(sc, dl, ssem)
        sr_start(ci, sc, swsem)
        start_dscatter()

    def pair0_body(t, _):
        c0 = 2 * t
        do_chunk0(c0, si0, dd0, ix0, xs0, xq0, gx0, gr0, sc0, dl0, ss0, sw0,
                  si1, dd1, ix1, xs1, xq1, gx1, gr1)
        do_chunk0(c0 + 1, si1, dd1, ix1, xs1, xq1, gx1, gr1,
                  sc1, dl1, ss1, sw1,
                  si0, dd0, ix0, xs0, xq0, gx0, gr0)
        return 0

    lax.fori_loop(0, NCHUNK // 2 - 1, pair0_body, 0)
    do_chunk0(NCHUNK - 2, si0, dd0, ix0, xs0, xq0, gx0, gr0,
              sc0, dl0, ss0, sw0,
              si1, dd1, ix1, xs1, xq1, gx1, gr1)
    do_chunk0(NCHUNK - 1, si1, dd1, ix1, xs1, xq1, gx1, gr1,
              sc1, dl1, ss1, sw1,
              si0, dd0, ix0, xs0, xq0, gx0, gr0, last=True)
    wait_idx(NCHUNK - 1, si0, dd0, ix0)
    wait_idx(NCHUNK - 1, si1, dd1, ix1)
    wait_scatter(sc0, dl0, ss0)
    wait_scatter(sc1, dl1, ss1)
    sr_wait(NCHUNK - 1, sc0, sw0)
    sr_wait(NCHUNK - 1, sc1, sw1)
    wait_dscatter()
    plsc.subcore_barrier()

    copy_out_all(0)
    plsc.subcore_barrier()
    zero_rows(dnbuf)
    init_idx(dl0, HALF)
    init_idx(dl1, HALF)
    zero_accum(dnbuf)
    plsc.subcore_barrier()

    def start_read(ci, sc, sem):
        pltpu.async_copy(sr_hbm.at[pl.ds(base + ci * CHUNK, CHUNK)], sc, sem)

    def wait_read(ci, sc, sem):
        pltpu.make_async_copy(
            sr_hbm.at[pl.ds(base + ci * CHUNK, CHUNK)], sc, sem).wait()

    def start_idx1(ci, dd, sem):
        cc = jnp.minimum(ci, NCHUNK - 1)
        pltpu.async_copy(dst_hbm.at[pl.ds(base + cc * CHUNK, CHUNK)], dd, sem)

    def wait_idx1(ci, dd, sem):
        cc = jnp.minimum(ci, NCHUNK - 1)
        pltpu.make_async_copy(dst_hbm.at[pl.ds(base + cc * CHUNK, CHUNK)],
                              dd, sem).wait()

    def compute_dl(ci, dd, dl):
        def group_body(g, _):
            dv = dd[pl.ds(g * 16, 16)] - HALF
            mine = dv >= 0
            dl[pl.ds(g * 16, 16)] = jnp.where(mine, dv, HALF)
            return 0

        lax.fori_loop(0, CHUNK // 16, group_body, 0)

    start_scatter(sc0, dl0, ss0)
    start_scatter(sc1, dl1, ss1)
    start_idx1(0, dd0, ix0)
    start_idx1(1, dd1, ix1)
    wait_idx1(0, dd0, ix0)
    wait_scatter(sc0, dl0, ss0)
    start_read(0, sc0, gx0)

    def do_chunk1(ci, dd, isem, sc, dl, ssem, gsem,
                  ndd, nisem, nsc, ndl, nssem, ngsem, last=False):
        wait_read(ci, sc, gsem)
        if not last:
            wait_idx1(ci + 1, ndd, nisem)
            wait_scatter(nsc, ndl, nssem)
            start_read(ci + 1, nsc, ngsem)
        compute_dl(ci, dd, dl)
        start_idx1(ci + 2, dd, isem)
        start_scatter(sc, dl, ssem)

    def pair1_body(t, _):
        c0 = 2 * t
        do_chunk1(c0, dd0, ix0, sc0, dl0, ss0, gx0,
                  dd1, ix1, sc1, dl1, ss1, gx1)
        do_chunk1(c0 + 1, dd1, ix1, sc1, dl1, ss1, gx1,
                  dd0, ix0, sc0, dl0, ss0, gx0)
        return 0

    lax.fori_loop(0, NCHUNK // 2 - 1, pair1_body, 0)
    do_chunk1(NCHUNK - 2, dd0, ix0, sc0, dl0, ss0, gx0,
              dd1, ix1, sc1, dl1, ss1, gx1)
    do_chunk1(NCHUNK - 1, dd1, ix1, sc1, dl1, ss1, gx1,
              dd0, ix0, sc0, dl0, ss0, gx0, last=True)
    wait_idx1(NCHUNK - 1, dd0, ix0)
    wait_idx1(NCHUNK - 1, dd1, ix1)
    wait_scatter(sc0, dl0, ss0)
    wait_scatter(sc1, dl1, ss1)
    plsc.subcore_barrier()

    copy_out_all(HALF)

    @pl.when(jnp.logical_and(s < DEN_R // 8, c == 0))
    def _():
        pltpu.sync_copy(dacc.at[pl.ds(s * 8, 8)], d0_hbm.at[pl.ds(s * 8, 8)])

    @pl.when(jnp.logical_and(s < DEN_R // 8, c == 1))
    def _():
        pltpu.sync_copy(dacc.at[pl.ds(s * 8, 8)], d1_hbm.at[pl.ds(s * 8, 8)])


def kernel(x, edge_index, params):
    sl = jnp.arange(N, dtype=jnp.int32)
    pad = jnp.zeros((EP - E_TOT,), jnp.int32)
    src = jnp.concatenate([edge_index[0].astype(jnp.int32), sl, pad])
    dst = jnp.concatenate([edge_index[1].astype(jnp.int32), sl, pad])
    xp = jnp.pad(x, ((0, NPAD - N), (0, 0)))

    nl = len(params) // 4
    p0 = p1 = d0 = d1 = bprev = None
    for i in range(nl):
        wl = params[f"Wl{i}"]
        wr = params[f"Wr{i}"]
        att = params[f"att{i}"].reshape(1, D)
        b = params[f"b{i}"].reshape(1, D)
        if i == 0:
            xl, xr, bl, br = _mm_first(xp, wl, wr, att)
        else:
            xl, xr, bl, br = _combine_mm(p0, p1, d0, d1, bprev, wl, wr, att)
        p0, p1, d0, d1, _ = _sc_layer(xl, xr, params[f"att{i}"], bl, br,
                                      src, dst)
        bprev = b
    return _final_pool(p0, p1, d0, d1, bprev)
